# Initial kernel scaffold; baseline (speedup 1.0000x reference)
import jax
import jax.numpy as jnp
from jax.experimental import pallas as pl


def kernel(x, edge_attr, edge_index, target_index, batch, target_class, pre_W1, pre_b1, pre_g1, pre_be1, pre_W2, pre_b2, pre_g2, pre_be2, enc_W1, enc_b1, enc_g1, enc_be1, enc_W2, enc_b2, enc_g2, enc_be2, lin0_W, lin1_W, lin1_b, conv_bias, gru_Wih, gru_Whh, gru_bih, gru_bhh, ln_g, ln_b, s2s_Wih, s2s_Whh, s2s_bih, s2s_bhh, pr_W1, pr_b1, pr_g1, pr_be1, pr_W2, pr_b2, pr_g2, pr_be2, pr_W3, pr_b3):
    raise NotImplementedError("write your pallas kernel here")


# speedup vs baseline: 1.0496x; 1.0496x over previous
import functools

import jax
import jax.numpy as jnp
from jax import lax
from jax.experimental import pallas as pl
from jax.experimental.pallas import tpu as pltpu
from jax.experimental.pallas import tpu_sc as plsc

N_NODES = 10000
N_EDGES = 160000
NF = 128
EF = 16
DIM = 32
NG = 64
NPAIR = 1024
STEPS = 3
NOUT = 8
EPS = 1e-5

NW = 32
CH = 128
E_PAD = 163840
NCH_E = E_PAD // (NW * CH)
N_ACC = 10016


def _pre_body(x_ref, w1_ref, b1_ref, g1_ref, be1_ref, w2_ref, b2_ref,
              g2_ref, be2_ref, out_ref):
    x = x_ref[...]
    y = jnp.dot(x, w1_ref[...], preferred_element_type=jnp.float32) + b1_ref[...]
    mu = jnp.mean(y, axis=0, keepdims=True)
    var = jnp.mean((y - mu) ** 2, axis=0, keepdims=True)
    y = (y - mu) * (g1_ref[...] * lax.rsqrt(var + EPS)) + be1_ref[...]
    y = jnp.maximum(y, 0.0)
    z = jnp.dot(y, w2_ref[...], preferred_element_type=jnp.float32) + b2_ref[...]
    mu2 = jnp.mean(z, axis=0, keepdims=True)
    var2 = jnp.mean((z - mu2) ** 2, axis=0, keepdims=True)
    z = (z - mu2) * (g2_ref[...] * lax.rsqrt(var2 + EPS)) + be2_ref[...]
    out_ref[...] = jnp.maximum(z, 0.0)


def _enc_stats_body(ea_ref, w1_ref, b1_ref, sum_ref, sq_ref):
    @pl.when(pl.program_id(0) == 0)
    def _():
        sum_ref[...] = jnp.zeros_like(sum_ref)
        sq_ref[...] = jnp.zeros_like(sq_ref)
    y = jnp.dot(ea_ref[...], w1_ref[...], preferred_element_type=jnp.float32) + b1_ref[...]
    sum_ref[...] += jnp.sum(y, axis=0, keepdims=True)
    sq_ref[...] += jnp.sum(y * y, axis=0, keepdims=True)


def _enc_he_body(ea_ref, w1_ref, b1_ref, mu_ref, sc_ref, be_ref,
                 he_ref, hsum_ref, hh_ref):
    @pl.when(pl.program_id(0) == 0)
    def _():
        hsum_ref[...] = jnp.zeros_like(hsum_ref)
        hh_ref[...] = jnp.zeros_like(hh_ref)
    y = jnp.dot(ea_ref[...], w1_ref[...], preferred_element_type=jnp.float32) + b1_ref[...]
    he = jnp.maximum((y - mu_ref[...]) * sc_ref[...] + be_ref[...], 0.0)
    he_ref[...] = he
    hsum_ref[...] += jnp.sum(he, axis=0, keepdims=True)
    hh_ref[...] += lax.dot_general(he, he, (((0,), (0,)), ((), ())),
                                   preferred_element_type=jnp.float32,
                                   precision=lax.Precision.HIGHEST)


def _msg_body(he_ref, x0_ref, wcat_ref, amat_ref, lin0_ref, lin1_ref,
              lin1b_ref, msg_ref):
    x0 = x0_ref[...]
    he = he_ref[...]
    et = x0.shape[0]
    u = jnp.dot(x0, wcat_ref[...], preferred_element_type=jnp.float32)
    x1 = jnp.dot(x0, amat_ref[...], preferred_element_type=jnp.float32)
    for k in range(DIM):
        x1 = x1 + he[:, k:k + 1] * u[:, k * DIM:(k + 1) * DIM]
    coeff = jax.nn.sigmoid(
        jnp.dot(x0, lin0_ref[...], preferred_element_type=jnp.float32)
        + jnp.dot(x1, lin1_ref[...], preferred_element_type=jnp.float32)
        + lin1b_ref[...])
    msg_ref[...] = x1 + coeff * (x0 - x1)


def _inv_cnt_body(cp_ref, inv_ref):
    cnt = cp_ref[0] + cp_ref[1]
    inv_ref[...] = 1.0 / jnp.maximum(cnt, 1.0)


def _gru_body(part_ref, inv_ref, cb_ref, h_ref, wih_ref, whh_ref, bih_ref,
              bhh_ref, lng_ref, lnb_ref, hout_ref, out_ref):
    agg = (part_ref[0] + part_ref[1]) * inv_ref[...] + cb_ref[...]
    m = jnp.maximum(agg, 0.0)
    h = h_ref[...]
    gi = jnp.dot(m, wih_ref[...], preferred_element_type=jnp.float32) + bih_ref[...]
    gh = jnp.dot(h, whh_ref[...], preferred_element_type=jnp.float32) + bhh_ref[...]
    r = jax.nn.sigmoid(gi[:, :DIM] + gh[:, :DIM])
    z = jax.nn.sigmoid(gi[:, DIM:2 * DIM] + gh[:, DIM:2 * DIM])
    n = jnp.tanh(gi[:, 2 * DIM:] + r * gh[:, 2 * DIM:])
    h2 = (1.0 - z) * n + z * h
    hout_ref[...] = h2
    mu = jnp.mean(h2, axis=-1, keepdims=True)
    var = jnp.mean((h2 - mu) ** 2, axis=-1, keepdims=True)
    out_ref[...] = (h2 - mu) * lax.rsqrt(var + EPS) * lng_ref[...] + lnb_ref[...]


def _s2s_body(out_ref, batch_ref, wih_ref, whh_ref, bih_ref, bhh_ref,
              qb_ref):
    out = out_ref[...]
    b = batch_ref[...]
    giota = lax.broadcasted_iota(jnp.int32, (N_NODES, NG), 1)
    maskb = giota == b
    maskt = maskb.astype(jnp.float32)
    q = jnp.zeros((NG, 2 * DIM), jnp.float32)
    hh = jnp.zeros((NG, DIM), jnp.float32)
    cc = jnp.zeros((NG, DIM), jnp.float32)
    for _ in range(STEPS):
        g = (jnp.dot(q, wih_ref[...], preferred_element_type=jnp.float32)
             + bih_ref[...]
             + jnp.dot(hh, whh_ref[...], preferred_element_type=jnp.float32)
             + bhh_ref[...])
        i = jax.nn.sigmoid(g[:, :DIM])
        f = jax.nn.sigmoid(g[:, DIM:2 * DIM])
        gg = jnp.tanh(g[:, 2 * DIM:3 * DIM])
        o = jax.nn.sigmoid(g[:, 3 * DIM:])
        cc = f * cc + i * gg
        hh = o * jnp.tanh(cc)
        hb = jnp.dot(maskt, hh, preferred_element_type=jnp.float32)
        e = jnp.sum(out * hb, axis=-1, keepdims=True)
        em = jnp.max(jnp.where(maskb, e, -1e30), axis=0, keepdims=True)
        emb = jnp.sum(maskt * em, axis=-1, keepdims=True)
        a = jnp.exp(e - emb)
        asum = jnp.sum(maskt * a, axis=0, keepdims=True)
        asb = jnp.sum(maskt * asum, axis=-1, keepdims=True)
        an = a / (asb + 1e-16)
        r = lax.dot_general(maskt, an * out, (((0,), (0,)), ((), ())),
                            preferred_element_type=jnp.float32)
        q = jnp.concatenate([hh, r], axis=-1)
    qb_ref[...] = jnp.dot(maskt, q, preferred_element_type=jnp.float32)


def _head_body(g2_ref, tc_ref, w1_ref, b1_ref, g1_ref, be1_ref, w2_ref,
               b2_ref, g2w_ref, be2_ref, w3_ref, b3_ref, res_ref):
    node0 = g2_ref[:NPAIR, :DIM]
    s2s0 = g2_ref[:NPAIR, DIM:]
    node1 = g2_ref[NPAIR:, :DIM]
    feat = jnp.concatenate([node0, node1, s2s0], axis=-1)

    def ln_relu(v, g, be):
        mu = jnp.mean(v, axis=-1, keepdims=True)
        var = jnp.mean((v - mu) ** 2, axis=-1, keepdims=True)
        return jnp.maximum((v - mu) * lax.rsqrt(var + EPS) * g + be, 0.0)

    z = ln_relu(jnp.dot(feat, w1_ref[...], preferred_element_type=jnp.float32)
                + b1_ref[...], g1_ref[...], be1_ref[...])
    z = ln_relu(jnp.dot(z, w2_ref[...], preferred_element_type=jnp.float32)
                + b2_ref[...], g2w_ref[...], be2_ref[...])
    pred = jnp.dot(z, w3_ref[...], preferred_element_type=jnp.float32) + b3_ref[...]
    sel = lax.broadcasted_iota(jnp.int32, (NPAIR, NOUT), 1) == tc_ref[...]
    res_ref[...] = jnp.sum(jnp.where(sel, pred, 0.0), axis=-1, keepdims=True)


def _sc_gather(table, idxp, nch, ch, d, nround=1):
    cpr = nch // nround
    mesh = plsc.VectorSubcoreMesh(core_axis_name="c", subcore_axis_name="s")

    @functools.partial(
        pl.kernel, mesh=mesh,
        out_type=jax.ShapeDtypeStruct((NW, nch * ch, d), jnp.float32),
        compiler_params=pltpu.CompilerParams(use_tc_tiling_on_sc=False),
        scratch_types=[
            pltpu.VMEM((nch, ch), jnp.int32),
            pltpu.VMEM((cpr * ch, d), jnp.float32),
            pltpu.SemaphoreType.DMA,
        ],
    )
    def k(table_hbm, idx_hbm, out_hbm, idx_v, rows_v, sem):
        c = lax.axis_index("c")
        s = lax.axis_index("s")
        w = s * 2 + c
        pltpu.sync_copy(idx_hbm.at[w], idx_v)
        for r in range(nround):
            copies = []
            for j in range(cpr):
                copies.append(pltpu.async_copy(
                    table_hbm.at[idx_v.at[r * cpr + j]],
                    rows_v.at[pl.ds(j * ch, ch)], sem))
            for cp in copies:
                cp.wait()
            pltpu.sync_copy(rows_v,
                            out_hbm.at[w].at[pl.ds(r * cpr * ch, cpr * ch)])

    return k(table, idxp)


def _sc_scatter_add(msgp, idxp, zeros_acc, nch, ch, d, nround=1):
    cpr = nch // nround
    mesh = plsc.VectorSubcoreMesh(core_axis_name="c", subcore_axis_name="s")
    zrows = N_ACC // 16

    @functools.partial(
        pl.kernel, mesh=mesh,
        out_type=jax.ShapeDtypeStruct((2, N_ACC, d), jnp.float32),
        compiler_params=pltpu.CompilerParams(use_tc_tiling_on_sc=False),
        scratch_types=[
            pltpu.VMEM((nch, ch), jnp.int32),
            pltpu.VMEM((cpr * ch, d), jnp.float32),
            pltpu.VMEM_SHARED((N_ACC, d), jnp.float32),
            pltpu.SemaphoreType.DMA,
        ],
    )
    def k(msg_hbm, idx_hbm, z_hbm, out_hbm, idx_v, msg_v, acc_sh, sem):
        c = lax.axis_index("c")
        s = lax.axis_index("s")
        w = s * 2 + c
        pltpu.sync_copy(z_hbm.at[pl.ds(s * zrows, zrows)],
                        acc_sh.at[pl.ds(s * zrows, zrows)])
        plsc.subcore_barrier()
        pltpu.sync_copy(idx_hbm.at[w], idx_v)
        for r in range(nround):
            pltpu.sync_copy(
                msg_hbm.at[w].at[pl.ds(r * cpr * ch, cpr * ch)], msg_v)
            for j in range(cpr):
                pltpu.sync_copy(msg_v.at[pl.ds(j * ch, ch)],
                                acc_sh.at[idx_v.at[r * cpr + j]], add=True)
        plsc.subcore_barrier()
        pltpu.sync_copy(acc_sh.at[pl.ds(s * zrows, zrows)],
                        out_hbm.at[c].at[pl.ds(s * zrows, zrows)])

    return k(msgp, idxp, zeros_acc)


def kernel(x, edge_attr, edge_index, target_index, batch, target_class, pre_W1, pre_b1, pre_g1, pre_be1, pre_W2, pre_b2, pre_g2, pre_be2, enc_W1, enc_b1, enc_g1, enc_be1, enc_W2, enc_b2, enc_g2, enc_be2, lin0_W, lin1_W, lin1_b, conv_bias, gru_Wih, gru_Whh, gru_bih, gru_bhh, ln_g, ln_b, s2s_Wih, s2s_Whh, s2s_bih, s2s_bhh, pr_W1, pr_b1, pr_g1, pr_be1, pr_W2, pr_b2, pr_g2, pr_be2, pr_W3, pr_b3):
    f32 = jnp.float32
    r2 = lambda v: v.reshape(1, -1).astype(f32)
    src = edge_index[0].astype(jnp.int32)
    dst = edge_index[1].astype(jnp.int32)

    out0 = pl.pallas_call(
        _pre_body,
        out_shape=jax.ShapeDtypeStruct((N_NODES, DIM), f32),
    )(x, pre_W1, r2(pre_b1), r2(pre_g1), r2(pre_be1),
      pre_W2, r2(pre_b2), r2(pre_g2), r2(pre_be2))

    ET2 = 8000
    T2 = N_EDGES // ET2
    sum1, sq1 = pl.pallas_call(
        _enc_stats_body,
        grid=(T2,),
        in_specs=[pl.BlockSpec((ET2, EF), lambda i: (i, 0)),
                  pl.BlockSpec((EF, DIM), lambda i: (0, 0)),
                  pl.BlockSpec((1, DIM), lambda i: (0, 0))],
        out_specs=[pl.BlockSpec((1, DIM), lambda i: (0, 0)),
                   pl.BlockSpec((1, DIM), lambda i: (0, 0))],
        out_shape=[jax.ShapeDtypeStruct((1, DIM), f32),
                   jax.ShapeDtypeStruct((1, DIM), f32)],
    )(edge_attr, enc_W1, r2(enc_b1))
    mu1 = sum1 / N_EDGES
    var1 = sq1 / N_EDGES - mu1 * mu1
    sc1 = r2(enc_g1) * lax.rsqrt(var1 + EPS)

    he, hsum, hth = pl.pallas_call(
        _enc_he_body,
        grid=(T2,),
        in_specs=[pl.BlockSpec((ET2, EF), lambda i: (i, 0)),
                  pl.BlockSpec((EF, DIM), lambda i: (0, 0)),
                  pl.BlockSpec((1, DIM), lambda i: (0, 0)),
                  pl.BlockSpec((1, DIM), lambda i: (0, 0)),
                  pl.BlockSpec((1, DIM), lambda i: (0, 0)),
                  pl.BlockSpec((1, DIM), lambda i: (0, 0))],
        out_specs=[pl.BlockSpec((ET2, DIM), lambda i: (i, 0)),
                   pl.BlockSpec((1, DIM), lambda i: (0, 0)),
                   pl.BlockSpec((DIM, DIM), lambda i: (0, 0))],
        out_shape=[jax.ShapeDtypeStruct((N_EDGES, DIM), f32),
                   jax.ShapeDtypeStruct((1, DIM), f32),
                   jax.ShapeDtypeStruct((DIM, DIM), f32)],
    )(edge_attr, enc_W1, r2(enc_b1), mu1, sc1, r2(enc_be1))

    hi = lax.Precision.HIGHEST
    mh = hsum / N_EDGES
    cov = (hth / N_EDGES
           - jnp.dot(mh.T, mh, precision=hi))
    mu2 = jnp.dot(mh, enc_W2, precision=hi) + enc_b2[None, :]
    var2 = jnp.sum(enc_W2 * jnp.dot(cov, enc_W2, precision=hi),
                   axis=0, keepdims=True)
    s2 = enc_g2[None, :] * lax.rsqrt(var2 + EPS)
    a_vec = (enc_b2[None, :] - mu2) * s2 + enc_be2[None, :]
    w2s = enc_W2 * s2
    wcat = w2s.reshape(DIM, DIM, DIM).transpose(1, 0, 2).reshape(DIM, DIM * DIM)
    a_mat = a_vec.reshape(DIM, DIM)

    src_p = jnp.pad(src, (0, E_PAD - N_EDGES)).reshape(NW, NCH_E, CH)
    dst_p = jnp.pad(dst, (0, E_PAD - N_EDGES),
                    constant_values=N_NODES).reshape(NW, NCH_E, CH)
    hep = jnp.pad(he, ((0, E_PAD - N_EDGES), (0, 0)))
    zeros_acc = jnp.zeros((N_ACC, DIM), f32)
    onesp = jnp.zeros((E_PAD, DIM), f32).at[:N_EDGES].set(1.0)

    cnt_part = _sc_scatter_add(onesp.reshape(NW, NCH_E * CH, DIM), dst_p,
                               zeros_acc, NCH_E, CH, DIM, nround=2)
    inv_cnt = pl.pallas_call(
        _inv_cnt_body,
        out_shape=jax.ShapeDtypeStruct((N_NODES, DIM), f32),
    )(cnt_part[:, :N_NODES, :])

    ET3 = 2048
    T3 = E_PAD // ET3
    h = out0
    out = out0
    wihT = gru_Wih.T
    whhT = gru_Whh.T
    for _ in range(STEPS):
        xj0 = _sc_gather(out, src_p, NCH_E, CH, DIM, nround=2).reshape(E_PAD, DIM)
        msgp = pl.pallas_call(
            _msg_body,
            grid=(T3,),
            in_specs=[pl.BlockSpec((ET3, DIM), lambda i: (i, 0)),
                      pl.BlockSpec((ET3, DIM), lambda i: (i, 0)),
                      pl.BlockSpec((DIM, DIM * DIM), lambda i: (0, 0)),
                      pl.BlockSpec((DIM, DIM), lambda i: (0, 0)),
                      pl.BlockSpec((DIM, DIM), lambda i: (0, 0)),
                      pl.BlockSpec((DIM, DIM), lambda i: (0, 0)),
                      pl.BlockSpec((1, DIM), lambda i: (0, 0))],
            out_specs=pl.BlockSpec((ET3, DIM), lambda i: (i, 0)),
            out_shape=jax.ShapeDtypeStruct((E_PAD, DIM), f32),
        )(hep, xj0, wcat, a_mat, lin0_W, lin1_W, r2(lin1_b))
        part = _sc_scatter_add(msgp.reshape(NW, NCH_E * CH, DIM), dst_p,
                               zeros_acc, NCH_E, CH, DIM, nround=2)
        h, out = pl.pallas_call(
            _gru_body,
            out_shape=[jax.ShapeDtypeStruct((N_NODES, DIM), f32),
                       jax.ShapeDtypeStruct((N_NODES, DIM), f32)],
        )(part[:, :N_NODES, :], inv_cnt, r2(conv_bias), h, wihT, whhT,
          r2(gru_bih), r2(gru_bhh), r2(ln_g), r2(ln_b))

    qb = pl.pallas_call(
        _s2s_body,
        out_shape=jax.ShapeDtypeStruct((N_NODES, 2 * DIM), f32),
    )(out, batch.astype(jnp.int32).reshape(N_NODES, 1), s2s_Wih.T, s2s_Whh.T,
      r2(s2s_bih), r2(s2s_bhh))

    atom0 = target_index[0].astype(jnp.int32)
    atom1 = target_index[1].astype(jnp.int32)
    table = jnp.concatenate([out, qb], axis=-1)
    pair_idx = jnp.concatenate([atom0, atom1]).reshape(NW, 1, 2 * NPAIR // NW)
    g2 = _sc_gather(table, pair_idx, 1, 2 * NPAIR // NW, 3 * DIM)
    g2 = g2.reshape(2 * NPAIR, 3 * DIM)

    res = pl.pallas_call(
        _head_body,
        out_shape=jax.ShapeDtypeStruct((NPAIR, 1), f32),
    )(g2, target_class.astype(jnp.int32).reshape(NPAIR, 1),
      pr_W1, r2(pr_b1), r2(pr_g1), r2(pr_be1),
      pr_W2, r2(pr_b2), r2(pr_g2), r2(pr_be2), pr_W3, r2(pr_b3))
    return res.reshape(NPAIR)


# speedup vs baseline: 2.4204x; 2.3060x over previous
import functools

import jax
import jax.numpy as jnp
from jax import lax
from jax.experimental import pallas as pl
from jax.experimental.pallas import tpu as pltpu
from jax.experimental.pallas import tpu_sc as plsc

N_NODES = 10000
N_EDGES = 160000
NF = 128
EF = 16
DIM = 32
NG = 64
NPAIR = 1024
STEPS = 3
NOUT = 8
EPS = 1e-5

NW = 32
CH = 128
E_PAD = 163840
NCH_E = E_PAD // (NW * CH)
N_ACC = 10016


def _pre_body(x_ref, w1_ref, b1_ref, g1_ref, be1_ref, w2_ref, b2_ref,
              g2_ref, be2_ref, out_ref):
    x = x_ref[...]
    y = jnp.dot(x, w1_ref[...], preferred_element_type=jnp.float32) + b1_ref[...]
    mu = jnp.mean(y, axis=0, keepdims=True)
    var = jnp.mean((y - mu) ** 2, axis=0, keepdims=True)
    y = (y - mu) * (g1_ref[...] * lax.rsqrt(var + EPS)) + be1_ref[...]
    y = jnp.maximum(y, 0.0)
    z = jnp.dot(y, w2_ref[...], preferred_element_type=jnp.float32) + b2_ref[...]
    mu2 = jnp.mean(z, axis=0, keepdims=True)
    var2 = jnp.mean((z - mu2) ** 2, axis=0, keepdims=True)
    z = (z - mu2) * (g2_ref[...] * lax.rsqrt(var2 + EPS)) + be2_ref[...]
    out_ref[...] = jnp.maximum(z, 0.0)


def _enc_stats_body(ea_ref, w1_ref, b1_ref, sum_ref, sq_ref):
    @pl.when(pl.program_id(0) == 0)
    def _():
        sum_ref[...] = jnp.zeros_like(sum_ref)
        sq_ref[...] = jnp.zeros_like(sq_ref)
    y = jnp.dot(ea_ref[...], w1_ref[...], preferred_element_type=jnp.float32) + b1_ref[...]
    sum_ref[...] += jnp.sum(y, axis=0, keepdims=True)
    sq_ref[...] += jnp.sum(y * y, axis=0, keepdims=True)


def _enc_he_body(ea_ref, w1_ref, b1_ref, mu_ref, sc_ref, be_ref,
                 he_ref, hsum_ref, hh_ref):
    @pl.when(pl.program_id(0) == 0)
    def _():
        hsum_ref[...] = jnp.zeros_like(hsum_ref)
        hh_ref[...] = jnp.zeros_like(hh_ref)
    y = jnp.dot(ea_ref[...], w1_ref[...], preferred_element_type=jnp.float32) + b1_ref[...]
    he = jnp.maximum((y - mu_ref[...]) * sc_ref[...] + be_ref[...], 0.0)
    he_ref[...] = he
    hsum_ref[...] += jnp.sum(he, axis=0, keepdims=True)
    hh_ref[...] += lax.dot_general(he, he, (((0,), (0,)), ((), ())),
                                   preferred_element_type=jnp.float32,
                                   precision=lax.Precision.HIGHEST)


def _msg_body(he_ref, x0_ref, wcat_ref, amat_ref, rmat_ref, smat_ref,
              lin0_ref, lin1_ref, lin1b_ref, msg_ref):
    x0 = x0_ref[...]
    he = he_ref[...]
    u = jnp.dot(x0, wcat_ref[...], preferred_element_type=jnp.float32)
    hrep = jnp.dot(he, rmat_ref[...], preferred_element_type=jnp.float32)
    x1 = jnp.dot(hrep * u, smat_ref[...], preferred_element_type=jnp.float32)
    x1 = x1 + jnp.dot(x0, amat_ref[...], preferred_element_type=jnp.float32)
    coeff = jax.nn.sigmoid(
        jnp.dot(x0, lin0_ref[...], preferred_element_type=jnp.float32)
        + jnp.dot(x1, lin1_ref[...], preferred_element_type=jnp.float32)
        + lin1b_ref[...])
    msg_ref[...] = x1 + coeff * (x0 - x1)


def _inv_cnt_body(cp_ref, inv_ref):
    cnt = cp_ref[0] + cp_ref[1]
    inv_ref[...] = 1.0 / jnp.maximum(cnt, 1.0)


def _gru_body(part_ref, inv_ref, cb_ref, h_ref, wih_ref, whh_ref, bih_ref,
              bhh_ref, lng_ref, lnb_ref, hout_ref, out_ref):
    agg = (part_ref[0] + part_ref[1]) * inv_ref[...] + cb_ref[...]
    m = jnp.maximum(agg, 0.0)
    h = h_ref[...]
    gi = jnp.dot(m, wih_ref[...], preferred_element_type=jnp.float32) + bih_ref[...]
    gh = jnp.dot(h, whh_ref[...], preferred_element_type=jnp.float32) + bhh_ref[...]
    r = jax.nn.sigmoid(gi[:, :DIM] + gh[:, :DIM])
    z = jax.nn.sigmoid(gi[:, DIM:2 * DIM] + gh[:, DIM:2 * DIM])
    n = jnp.tanh(gi[:, 2 * DIM:] + r * gh[:, 2 * DIM:])
    h2 = (1.0 - z) * n + z * h
    hout_ref[...] = h2
    mu = jnp.mean(h2, axis=-1, keepdims=True)
    var = jnp.mean((h2 - mu) ** 2, axis=-1, keepdims=True)
    out_ref[...] = (h2 - mu) * lax.rsqrt(var + EPS) * lng_ref[...] + lnb_ref[...]


def _s2s_body(out_ref, batch_ref, wih_ref, whh_ref, bih_ref, bhh_ref,
              qb_ref):
    out = out_ref[...]
    b = batch_ref[...]
    giota = lax.broadcasted_iota(jnp.int32, (N_NODES, NG), 1)
    maskb = giota == b
    maskt = maskb.astype(jnp.float32)
    q = jnp.zeros((NG, 2 * DIM), jnp.float32)
    hh = jnp.zeros((NG, DIM), jnp.float32)
    cc = jnp.zeros((NG, DIM), jnp.float32)
    for _ in range(STEPS):
        g = (jnp.dot(q, wih_ref[...], preferred_element_type=jnp.float32)
             + bih_ref[...]
             + jnp.dot(hh, whh_ref[...], preferred_element_type=jnp.float32)
             + bhh_ref[...])
        i = jax.nn.sigmoid(g[:, :DIM])
        f = jax.nn.sigmoid(g[:, DIM:2 * DIM])
        gg = jnp.tanh(g[:, 2 * DIM:3 * DIM])
        o = jax.nn.sigmoid(g[:, 3 * DIM:])
        cc = f * cc + i * gg
        hh = o * jnp.tanh(cc)
        hb = jnp.dot(maskt, hh, preferred_element_type=jnp.float32)
        e = jnp.sum(out * hb, axis=-1, keepdims=True)
        em = jnp.max(jnp.where(maskb, e, -1e30), axis=0, keepdims=True)
        emb = jnp.sum(maskt * em, axis=-1, keepdims=True)
        a = jnp.exp(e - emb)
        asum = jnp.sum(maskt * a, axis=0, keepdims=True)
        asb = jnp.sum(maskt * asum, axis=-1, keepdims=True)
        an = a / (asb + 1e-16)
        r = lax.dot_general(maskt, an * out, (((0,), (0,)), ((), ())),
                            preferred_element_type=jnp.float32)
        q = jnp.concatenate([hh, r], axis=-1)
    qb_ref[...] = jnp.dot(maskt, q, preferred_element_type=jnp.float32)


def _head_body(g2_ref, tc_ref, w1_ref, b1_ref, g1_ref, be1_ref, w2_ref,
               b2_ref, g2w_ref, be2_ref, w3_ref, b3_ref, res_ref):
    node0 = g2_ref[:NPAIR, :DIM]
    s2s0 = g2_ref[:NPAIR, DIM:]
    node1 = g2_ref[NPAIR:, :DIM]
    feat = jnp.concatenate([node0, node1, s2s0], axis=-1)

    def ln_relu(v, g, be):
        mu = jnp.mean(v, axis=-1, keepdims=True)
        var = jnp.mean((v - mu) ** 2, axis=-1, keepdims=True)
        return jnp.maximum((v - mu) * lax.rsqrt(var + EPS) * g + be, 0.0)

    z = ln_relu(jnp.dot(feat, w1_ref[...], preferred_element_type=jnp.float32)
                + b1_ref[...], g1_ref[...], be1_ref[...])
    z = ln_relu(jnp.dot(z, w2_ref[...], preferred_element_type=jnp.float32)
                + b2_ref[...], g2w_ref[...], be2_ref[...])
    pred = jnp.dot(z, w3_ref[...], preferred_element_type=jnp.float32) + b3_ref[...]
    sel = lax.broadcasted_iota(jnp.int32, (NPAIR, NOUT), 1) == tc_ref[...]
    res_ref[...] = jnp.sum(jnp.where(sel, pred, 0.0), axis=-1, keepdims=True)


def _sc_gather(table, idxp, nch, ch, d, nround=1):
    cpr = nch // nround
    mesh = plsc.VectorSubcoreMesh(core_axis_name="c", subcore_axis_name="s")

    @functools.partial(
        pl.kernel, mesh=mesh,
        out_type=jax.ShapeDtypeStruct((NW, nch * ch, d), jnp.float32),
        compiler_params=pltpu.CompilerParams(use_tc_tiling_on_sc=False),
        scratch_types=[
            pltpu.VMEM((nch, ch), jnp.int32),
            pltpu.VMEM((cpr * ch, d), jnp.float32),
            pltpu.SemaphoreType.DMA,
        ],
    )
    def k(table_hbm, idx_hbm, out_hbm, idx_v, rows_v, sem):
        c = lax.axis_index("c")
        s = lax.axis_index("s")
        w = s * 2 + c
        pltpu.sync_copy(idx_hbm.at[w], idx_v)
        for r in range(nround):
            copies = []
            for j in range(cpr):
                copies.append(pltpu.async_copy(
                    table_hbm.at[idx_v.at[r * cpr + j]],
                    rows_v.at[pl.ds(j * ch, ch)], sem))
            for cp in copies:
                cp.wait()
            pltpu.sync_copy(rows_v,
                            out_hbm.at[w].at[pl.ds(r * cpr * ch, cpr * ch)])

    return k(table, idxp)


def _sc_scatter_add(msgp, idxp, zeros_acc, nch, ch, d, nround=1):
    cpr = nch // nround
    mesh = plsc.VectorSubcoreMesh(core_axis_name="c", subcore_axis_name="s")
    zrows = N_ACC // 16

    @functools.partial(
        pl.kernel, mesh=mesh,
        out_type=jax.ShapeDtypeStruct((2, N_ACC, d), jnp.float32),
        compiler_params=pltpu.CompilerParams(use_tc_tiling_on_sc=False),
        scratch_types=[
            pltpu.VMEM((nch, ch), jnp.int32),
            pltpu.VMEM((cpr * ch, d), jnp.float32),
            pltpu.VMEM_SHARED((N_ACC, d), jnp.float32),
            pltpu.SemaphoreType.DMA,
        ],
    )
    def k(msg_hbm, idx_hbm, z_hbm, out_hbm, idx_v, msg_v, acc_sh, sem):
        c = lax.axis_index("c")
        s = lax.axis_index("s")
        w = s * 2 + c
        pltpu.sync_copy(z_hbm.at[pl.ds(s * zrows, zrows)],
                        acc_sh.at[pl.ds(s * zrows, zrows)])
        plsc.subcore_barrier()
        pltpu.sync_copy(idx_hbm.at[w], idx_v)
        for r in range(nround):
            pltpu.sync_copy(
                msg_hbm.at[w].at[pl.ds(r * cpr * ch, cpr * ch)], msg_v)
            for j in range(cpr):
                pltpu.sync_copy(msg_v.at[pl.ds(j * ch, ch)],
                                acc_sh.at[idx_v.at[r * cpr + j]], add=True)
        plsc.subcore_barrier()
        pltpu.sync_copy(acc_sh.at[pl.ds(s * zrows, zrows)],
                        out_hbm.at[c].at[pl.ds(s * zrows, zrows)])

    return k(msgp, idxp, zeros_acc)


def kernel(x, edge_attr, edge_index, target_index, batch, target_class, pre_W1, pre_b1, pre_g1, pre_be1, pre_W2, pre_b2, pre_g2, pre_be2, enc_W1, enc_b1, enc_g1, enc_be1, enc_W2, enc_b2, enc_g2, enc_be2, lin0_W, lin1_W, lin1_b, conv_bias, gru_Wih, gru_Whh, gru_bih, gru_bhh, ln_g, ln_b, s2s_Wih, s2s_Whh, s2s_bih, s2s_bhh, pr_W1, pr_b1, pr_g1, pr_be1, pr_W2, pr_b2, pr_g2, pr_be2, pr_W3, pr_b3):
    f32 = jnp.float32
    r2 = lambda v: v.reshape(1, -1).astype(f32)
    src = edge_index[0].astype(jnp.int32)
    dst = edge_index[1].astype(jnp.int32)

    out0 = pl.pallas_call(
        _pre_body,
        out_shape=jax.ShapeDtypeStruct((N_NODES, DIM), f32),
    )(x, pre_W1, r2(pre_b1), r2(pre_g1), r2(pre_be1),
      pre_W2, r2(pre_b2), r2(pre_g2), r2(pre_be2))

    ET2 = 8000
    T2 = N_EDGES // ET2
    sum1, sq1 = pl.pallas_call(
        _enc_stats_body,
        grid=(T2,),
        in_specs=[pl.BlockSpec((ET2, EF), lambda i: (i, 0)),
                  pl.BlockSpec((EF, DIM), lambda i: (0, 0)),
                  pl.BlockSpec((1, DIM), lambda i: (0, 0))],
        out_specs=[pl.BlockSpec((1, DIM), lambda i: (0, 0)),
                   pl.BlockSpec((1, DIM), lambda i: (0, 0))],
        out_shape=[jax.ShapeDtypeStruct((1, DIM), f32),
                   jax.ShapeDtypeStruct((1, DIM), f32)],
    )(edge_attr, enc_W1, r2(enc_b1))
    mu1 = sum1 / N_EDGES
    var1 = sq1 / N_EDGES - mu1 * mu1
    sc1 = r2(enc_g1) * lax.rsqrt(var1 + EPS)

    he, hsum, hth = pl.pallas_call(
        _enc_he_body,
        grid=(T2,),
        in_specs=[pl.BlockSpec((ET2, EF), lambda i: (i, 0)),
                  pl.BlockSpec((EF, DIM), lambda i: (0, 0)),
                  pl.BlockSpec((1, DIM), lambda i: (0, 0)),
                  pl.BlockSpec((1, DIM), lambda i: (0, 0)),
                  pl.BlockSpec((1, DIM), lambda i: (0, 0)),
                  pl.BlockSpec((1, DIM), lambda i: (0, 0))],
        out_specs=[pl.BlockSpec((ET2, DIM), lambda i: (i, 0)),
                   pl.BlockSpec((1, DIM), lambda i: (0, 0)),
                   pl.BlockSpec((DIM, DIM), lambda i: (0, 0))],
        out_shape=[jax.ShapeDtypeStruct((N_EDGES, DIM), f32),
                   jax.ShapeDtypeStruct((1, DIM), f32),
                   jax.ShapeDtypeStruct((DIM, DIM), f32)],
    )(edge_attr, enc_W1, r2(enc_b1), mu1, sc1, r2(enc_be1))

    hi = lax.Precision.HIGHEST
    mh = hsum / N_EDGES
    cov = (hth / N_EDGES
           - jnp.dot(mh.T, mh, precision=hi))
    mu2 = jnp.dot(mh, enc_W2, precision=hi) + enc_b2[None, :]
    var2 = jnp.sum(enc_W2 * jnp.dot(cov, enc_W2, precision=hi),
                   axis=0, keepdims=True)
    s2 = enc_g2[None, :] * lax.rsqrt(var2 + EPS)
    a_vec = (enc_b2[None, :] - mu2) * s2 + enc_be2[None, :]
    w2s = enc_W2 * s2
    wcat = w2s.reshape(DIM, DIM, DIM).transpose(1, 0, 2).reshape(DIM, DIM * DIM)
    a_mat = a_vec.reshape(DIM, DIM)
    kk = jnp.arange(DIM * DIM, dtype=jnp.int32)
    r_mat = (jnp.arange(DIM, dtype=jnp.int32)[:, None] == kk[None, :] // DIM)
    r_mat = r_mat.astype(f32)
    s_mat = (kk[:, None] % DIM ==
             jnp.arange(DIM, dtype=jnp.int32)[None, :]).astype(f32)

    src_p = jnp.pad(src, (0, E_PAD - N_EDGES)).reshape(NW, NCH_E, CH)
    dst_p = jnp.pad(dst, (0, E_PAD - N_EDGES),
                    constant_values=N_NODES).reshape(NW, NCH_E, CH)
    hep = jnp.pad(he, ((0, E_PAD - N_EDGES), (0, 0)))
    zeros_acc = jnp.zeros((N_ACC, DIM), f32)
    onesp = jnp.zeros((E_PAD, DIM), f32).at[:N_EDGES].set(1.0)

    cnt_part = _sc_scatter_add(onesp.reshape(NW, NCH_E * CH, DIM), dst_p,
                               zeros_acc, NCH_E, CH, DIM, nround=2)
    inv_cnt = pl.pallas_call(
        _inv_cnt_body,
        out_shape=jax.ShapeDtypeStruct((N_NODES, DIM), f32),
    )(cnt_part[:, :N_NODES, :])

    ET3 = 2048
    T3 = E_PAD // ET3
    h = out0
    out = out0
    wihT = gru_Wih.T
    whhT = gru_Whh.T
    for _ in range(STEPS):
        xj0 = _sc_gather(out, src_p, NCH_E, CH, DIM, nround=2).reshape(E_PAD, DIM)
        msgp = pl.pallas_call(
            _msg_body,
            grid=(T3,),
            in_specs=[pl.BlockSpec((ET3, DIM), lambda i: (i, 0)),
                      pl.BlockSpec((ET3, DIM), lambda i: (i, 0)),
                      pl.BlockSpec((DIM, DIM * DIM), lambda i: (0, 0)),
                      pl.BlockSpec((DIM, DIM), lambda i: (0, 0)),
                      pl.BlockSpec((DIM, DIM * DIM), lambda i: (0, 0)),
                      pl.BlockSpec((DIM * DIM, DIM), lambda i: (0, 0)),
                      pl.BlockSpec((DIM, DIM), lambda i: (0, 0)),
                      pl.BlockSpec((DIM, DIM), lambda i: (0, 0)),
                      pl.BlockSpec((1, DIM), lambda i: (0, 0))],
            out_specs=pl.BlockSpec((ET3, DIM), lambda i: (i, 0)),
            out_shape=jax.ShapeDtypeStruct((E_PAD, DIM), f32),
        )(hep, xj0, wcat, a_mat, r_mat, s_mat, lin0_W, lin1_W, r2(lin1_b))
        part = _sc_scatter_add(msgp.reshape(NW, NCH_E * CH, DIM), dst_p,
                               zeros_acc, NCH_E, CH, DIM, nround=2)
        h, out = pl.pallas_call(
            _gru_body,
            out_shape=[jax.ShapeDtypeStruct((N_NODES, DIM), f32),
                       jax.ShapeDtypeStruct((N_NODES, DIM), f32)],
        )(part[:, :N_NODES, :], inv_cnt, r2(conv_bias), h, wihT, whhT,
          r2(gru_bih), r2(gru_bhh), r2(ln_g), r2(ln_b))

    qb = pl.pallas_call(
        _s2s_body,
        out_shape=jax.ShapeDtypeStruct((N_NODES, 2 * DIM), f32),
    )(out, batch.astype(jnp.int32).reshape(N_NODES, 1), s2s_Wih.T, s2s_Whh.T,
      r2(s2s_bih), r2(s2s_bhh))

    atom0 = target_index[0].astype(jnp.int32)
    atom1 = target_index[1].astype(jnp.int32)
    table = jnp.concatenate([out, qb], axis=-1)
    pair_idx = jnp.concatenate([atom0, atom1]).reshape(NW, 1, 2 * NPAIR // NW)
    g2 = _sc_gather(table, pair_idx, 1, 2 * NPAIR // NW, 3 * DIM)
    g2 = g2.reshape(2 * NPAIR, 3 * DIM)

    res = pl.pallas_call(
        _head_body,
        out_shape=jax.ShapeDtypeStruct((NPAIR, 1), f32),
    )(g2, target_class.astype(jnp.int32).reshape(NPAIR, 1),
      pr_W1, r2(pr_b1), r2(pr_g1), r2(pr_be1),
      pr_W2, r2(pr_b2), r2(pr_g2), r2(pr_be2), pr_W3, r2(pr_b3))
    return res.reshape(NPAIR)


# speedup vs baseline: 2.5071x; 1.0358x over previous
import functools

import jax
import jax.numpy as jnp
from jax import lax
from jax.experimental import pallas as pl
from jax.experimental.pallas import tpu as pltpu
from jax.experimental.pallas import tpu_sc as plsc

N_NODES = 10000
N_EDGES = 160000
NF = 128
EF = 16
DIM = 32
NG = 64
NPAIR = 1024
STEPS = 3
NOUT = 8
EPS = 1e-5

NW = 32
CH = 128
E_PAD = 163840
NCH_E = E_PAD // (NW * CH)
N_ACC = 10016


def _pre_body(x_ref, w1_ref, b1_ref, g1_ref, be1_ref, w2_ref, b2_ref,
              g2_ref, be2_ref, out_ref):
    x = x_ref[...]
    y = jnp.dot(x, w1_ref[...], preferred_element_type=jnp.float32) + b1_ref[...]
    mu = jnp.mean(y, axis=0, keepdims=True)
    var = jnp.mean((y - mu) ** 2, axis=0, keepdims=True)
    y = (y - mu) * (g1_ref[...] * lax.rsqrt(var + EPS)) + be1_ref[...]
    y = jnp.maximum(y, 0.0)
    z = jnp.dot(y, w2_ref[...], preferred_element_type=jnp.float32) + b2_ref[...]
    mu2 = jnp.mean(z, axis=0, keepdims=True)
    var2 = jnp.mean((z - mu2) ** 2, axis=0, keepdims=True)
    z = (z - mu2) * (g2_ref[...] * lax.rsqrt(var2 + EPS)) + be2_ref[...]
    out_ref[...] = jnp.maximum(z, 0.0)


def _enc_stats_body(ea_ref, w1_ref, b1_ref, sum_ref, sq_ref):
    @pl.when(pl.program_id(0) == 0)
    def _():
        sum_ref[...] = jnp.zeros_like(sum_ref)
        sq_ref[...] = jnp.zeros_like(sq_ref)
    y = jnp.dot(ea_ref[...], w1_ref[...], preferred_element_type=jnp.float32) + b1_ref[...]
    sum_ref[...] += jnp.sum(y, axis=0, keepdims=True)
    sq_ref[...] += jnp.sum(y * y, axis=0, keepdims=True)


def _enc_he_body(ea_ref, w1_ref, b1_ref, mu_ref, sc_ref, be_ref,
                 he_ref, hsum_ref, hh_ref):
    @pl.when(pl.program_id(0) == 0)
    def _():
        hsum_ref[...] = jnp.zeros_like(hsum_ref)
        hh_ref[...] = jnp.zeros_like(hh_ref)
    y = jnp.dot(ea_ref[...], w1_ref[...], preferred_element_type=jnp.float32) + b1_ref[...]
    he = jnp.maximum((y - mu_ref[...]) * sc_ref[...] + be_ref[...], 0.0)
    he_ref[...] = he
    hsum_ref[...] += jnp.sum(he, axis=0, keepdims=True)
    hh_ref[...] += lax.dot_general(he, he, (((0,), (0,)), ((), ())),
                                   preferred_element_type=jnp.float32,
                                   precision=lax.Precision.HIGHEST)


def _msg_body(he_ref, x0_ref, wcat_ref, amat_ref, rmat_ref, smat_ref,
              lin0_ref, lin1_ref, lin1b_ref, msg_ref):
    x0 = x0_ref[...]
    he = he_ref[...]
    u = jnp.dot(x0, wcat_ref[...], preferred_element_type=jnp.float32)
    hrep = jnp.dot(he, rmat_ref[...], preferred_element_type=jnp.float32)
    x1 = jnp.dot(hrep * u, smat_ref[...], preferred_element_type=jnp.float32)
    x1 = x1 + jnp.dot(x0, amat_ref[...], preferred_element_type=jnp.float32)
    coeff = jax.nn.sigmoid(
        jnp.dot(x0, lin0_ref[...], preferred_element_type=jnp.float32)
        + jnp.dot(x1, lin1_ref[...], preferred_element_type=jnp.float32)
        + lin1b_ref[...])
    msg_ref[...] = x1 + coeff * (x0 - x1)


def _gru_body(part_ref, cnt_ref, cb_ref, h_ref, wih_ref, whh_ref, bih_ref,
              bhh_ref, lng_ref, lnb_ref, hout_ref, out_ref):
    cnt = cnt_ref[0][:, :1] + cnt_ref[1][:, :1]
    inv = 1.0 / jnp.maximum(cnt, 1.0)
    agg = (part_ref[0] + part_ref[1]) * inv + cb_ref[...]
    m = jnp.maximum(agg, 0.0)
    h = h_ref[...]
    gi = jnp.dot(m, wih_ref[...], preferred_element_type=jnp.float32) + bih_ref[...]
    gh = jnp.dot(h, whh_ref[...], preferred_element_type=jnp.float32) + bhh_ref[...]
    r = jax.nn.sigmoid(gi[:, :DIM] + gh[:, :DIM])
    z = jax.nn.sigmoid(gi[:, DIM:2 * DIM] + gh[:, DIM:2 * DIM])
    n = jnp.tanh(gi[:, 2 * DIM:] + r * gh[:, 2 * DIM:])
    h2 = (1.0 - z) * n + z * h
    hout_ref[...] = h2
    mu = jnp.mean(h2, axis=-1, keepdims=True)
    var = jnp.mean((h2 - mu) ** 2, axis=-1, keepdims=True)
    out_ref[...] = (h2 - mu) * lax.rsqrt(var + EPS) * lng_ref[...] + lnb_ref[...]


def _s2s_body(out_ref, batch_ref, wih_ref, whh_ref, bih_ref, bhh_ref,
              tab_ref):
    out = out_ref[...]
    b = batch_ref[...]
    giota = lax.broadcasted_iota(jnp.int32, (N_NODES, NG), 1)
    maskb = giota == b
    maskt = maskb.astype(jnp.float32)
    q = jnp.zeros((NG, 2 * DIM), jnp.float32)
    hh = jnp.zeros((NG, DIM), jnp.float32)
    cc = jnp.zeros((NG, DIM), jnp.float32)
    for _ in range(STEPS):
        g = (jnp.dot(q, wih_ref[...], preferred_element_type=jnp.float32)
             + bih_ref[...]
             + jnp.dot(hh, whh_ref[...], preferred_element_type=jnp.float32)
             + bhh_ref[...])
        i = jax.nn.sigmoid(g[:, :DIM])
        f = jax.nn.sigmoid(g[:, DIM:2 * DIM])
        gg = jnp.tanh(g[:, 2 * DIM:3 * DIM])
        o = jax.nn.sigmoid(g[:, 3 * DIM:])
        cc = f * cc + i * gg
        hh = o * jnp.tanh(cc)
        hb = jnp.dot(maskt, hh, preferred_element_type=jnp.float32)
        e = jnp.sum(out * hb, axis=-1, keepdims=True)
        em = jnp.max(jnp.where(maskb, e, -1e30), axis=0, keepdims=True)
        emb = jnp.sum(maskt * em, axis=-1, keepdims=True)
        a = jnp.exp(e - emb)
        asum = jnp.sum(maskt * a, axis=0, keepdims=True)
        asb = jnp.sum(maskt * asum, axis=-1, keepdims=True)
        an = a / (asb + 1e-16)
        r = lax.dot_general(maskt, an * out, (((0,), (0,)), ((), ())),
                            preferred_element_type=jnp.float32)
        q = jnp.concatenate([hh, r], axis=-1)
    tab_ref[:, :DIM] = out
    tab_ref[:, DIM:] = jnp.dot(maskt, q, preferred_element_type=jnp.float32)


def _head_body(g2_ref, tc_ref, w1_ref, b1_ref, g1_ref, be1_ref, w2_ref,
               b2_ref, g2w_ref, be2_ref, w3_ref, b3_ref, res_ref):
    node0 = g2_ref[:NPAIR, :DIM]
    s2s0 = g2_ref[:NPAIR, DIM:]
    node1 = g2_ref[NPAIR:, :DIM]
    feat = jnp.concatenate([node0, node1, s2s0], axis=-1)

    def ln_relu(v, g, be):
        mu = jnp.mean(v, axis=-1, keepdims=True)
        var = jnp.mean((v - mu) ** 2, axis=-1, keepdims=True)
        return jnp.maximum((v - mu) * lax.rsqrt(var + EPS) * g + be, 0.0)

    z = ln_relu(jnp.dot(feat, w1_ref[...], preferred_element_type=jnp.float32)
                + b1_ref[...], g1_ref[...], be1_ref[...])
    z = ln_relu(jnp.dot(z, w2_ref[...], preferred_element_type=jnp.float32)
                + b2_ref[...], g2w_ref[...], be2_ref[...])
    pred = jnp.dot(z, w3_ref[...], preferred_element_type=jnp.float32) + b3_ref[...]
    sel = lax.broadcasted_iota(jnp.int32, (NPAIR, NOUT), 1) == tc_ref[...]
    res_ref[...] = jnp.sum(jnp.where(sel, pred, 0.0), axis=-1, keepdims=True)


def _sc_gather(table, idxp, nch, ch, d, nround=1):
    cpr = nch // nround
    mesh = plsc.VectorSubcoreMesh(core_axis_name="c", subcore_axis_name="s")

    @functools.partial(
        pl.kernel, mesh=mesh,
        out_type=jax.ShapeDtypeStruct((NW, nch * ch, d), jnp.float32),
        compiler_params=pltpu.CompilerParams(use_tc_tiling_on_sc=False),
        scratch_types=[
            pltpu.VMEM((nch, ch), jnp.int32),
            pltpu.VMEM((cpr * ch, d), jnp.float32),
            pltpu.SemaphoreType.DMA,
        ],
    )
    def k(table_hbm, idx_hbm, out_hbm, idx_v, rows_v, sem):
        c = lax.axis_index("c")
        s = lax.axis_index("s")
        w = s * 2 + c
        pltpu.sync_copy(idx_hbm.at[w], idx_v)
        for r in range(nround):
            copies = []
            for j in range(cpr):
                copies.append(pltpu.async_copy(
                    table_hbm.at[idx_v.at[r * cpr + j]],
                    rows_v.at[pl.ds(j * ch, ch)], sem))
            for cp in copies:
                cp.wait()
            pltpu.sync_copy(rows_v,
                            out_hbm.at[w].at[pl.ds(r * cpr * ch, cpr * ch)])

    return k(table, idxp)


def _sc_scatter_add(msgp, idxp, zeros_acc, nch, ch, d, nround=1):
    cpr = nch // nround
    mesh = plsc.VectorSubcoreMesh(core_axis_name="c", subcore_axis_name="s")
    zrows = N_ACC // 16

    @functools.partial(
        pl.kernel, mesh=mesh,
        out_type=jax.ShapeDtypeStruct((2, N_ACC, d), jnp.float32),
        compiler_params=pltpu.CompilerParams(use_tc_tiling_on_sc=False),
        scratch_types=[
            pltpu.VMEM((nch, ch), jnp.int32),
            pltpu.VMEM((2, cpr * ch, d), jnp.float32),
            pltpu.VMEM_SHARED((N_ACC, d), jnp.float32),
            pltpu.SemaphoreType.DMA,
        ],
    )
    def k(msg_hbm, idx_hbm, z_hbm, out_hbm, idx_v, msg_v, acc_sh, sem):
        c = lax.axis_index("c")
        s = lax.axis_index("s")
        w = s * 2 + c
        pltpu.sync_copy(z_hbm.at[pl.ds(s * zrows, zrows)],
                        acc_sh.at[pl.ds(s * zrows, zrows)])
        plsc.subcore_barrier()
        pltpu.sync_copy(idx_hbm.at[w], idx_v)
        pltpu.sync_copy(msg_hbm.at[w].at[pl.ds(0, cpr * ch)], msg_v.at[0])
        for r in range(nround):
            adds = []
            for j in range(cpr):
                adds.append(pltpu.async_copy(
                    msg_v.at[r % 2].at[pl.ds(j * ch, ch)],
                    acc_sh.at[idx_v.at[r * cpr + j]], sem, add=True))
            if r + 1 < nround:
                pltpu.sync_copy(
                    msg_hbm.at[w].at[pl.ds((r + 1) * cpr * ch, cpr * ch)],
                    msg_v.at[(r + 1) % 2])
            for cp in adds:
                cp.wait()
        plsc.subcore_barrier()
        pltpu.sync_copy(acc_sh.at[pl.ds(s * zrows, zrows)],
                        out_hbm.at[c].at[pl.ds(s * zrows, zrows)])

    return k(msgp, idxp, zeros_acc)


def kernel(x, edge_attr, edge_index, target_index, batch, target_class, pre_W1, pre_b1, pre_g1, pre_be1, pre_W2, pre_b2, pre_g2, pre_be2, enc_W1, enc_b1, enc_g1, enc_be1, enc_W2, enc_b2, enc_g2, enc_be2, lin0_W, lin1_W, lin1_b, conv_bias, gru_Wih, gru_Whh, gru_bih, gru_bhh, ln_g, ln_b, s2s_Wih, s2s_Whh, s2s_bih, s2s_bhh, pr_W1, pr_b1, pr_g1, pr_be1, pr_W2, pr_b2, pr_g2, pr_be2, pr_W3, pr_b3):
    f32 = jnp.float32
    r2 = lambda v: v.reshape(1, -1).astype(f32)
    src = edge_index[0].astype(jnp.int32)
    dst = edge_index[1].astype(jnp.int32)

    out0 = pl.pallas_call(
        _pre_body,
        out_shape=jax.ShapeDtypeStruct((N_NODES, DIM), f32),
    )(x, pre_W1, r2(pre_b1), r2(pre_g1), r2(pre_be1),
      pre_W2, r2(pre_b2), r2(pre_g2), r2(pre_be2))

    ET2 = 8000
    T2 = N_EDGES // ET2
    sum1, sq1 = pl.pallas_call(
        _enc_stats_body,
        grid=(T2,),
        in_specs=[pl.BlockSpec((ET2, EF), lambda i: (i, 0)),
                  pl.BlockSpec((EF, DIM), lambda i: (0, 0)),
                  pl.BlockSpec((1, DIM), lambda i: (0, 0))],
        out_specs=[pl.BlockSpec((1, DIM), lambda i: (0, 0)),
                   pl.BlockSpec((1, DIM), lambda i: (0, 0))],
        out_shape=[jax.ShapeDtypeStruct((1, DIM), f32),
                   jax.ShapeDtypeStruct((1, DIM), f32)],
    )(edge_attr, enc_W1, r2(enc_b1))
    mu1 = sum1 / N_EDGES
    var1 = sq1 / N_EDGES - mu1 * mu1
    sc1 = r2(enc_g1) * lax.rsqrt(var1 + EPS)

    he, hsum, hth = pl.pallas_call(
        _enc_he_body,
        grid=(T2,),
        in_specs=[pl.BlockSpec((ET2, EF), lambda i: (i, 0)),
                  pl.BlockSpec((EF, DIM), lambda i: (0, 0)),
                  pl.BlockSpec((1, DIM), lambda i: (0, 0)),
                  pl.BlockSpec((1, DIM), lambda i: (0, 0)),
                  pl.BlockSpec((1, DIM), lambda i: (0, 0)),
                  pl.BlockSpec((1, DIM), lambda i: (0, 0))],
        out_specs=[pl.BlockSpec((ET2, DIM), lambda i: (i, 0)),
                   pl.BlockSpec((1, DIM), lambda i: (0, 0)),
                   pl.BlockSpec((DIM, DIM), lambda i: (0, 0))],
        out_shape=[jax.ShapeDtypeStruct((N_EDGES, DIM), f32),
                   jax.ShapeDtypeStruct((1, DIM), f32),
                   jax.ShapeDtypeStruct((DIM, DIM), f32)],
    )(edge_attr, enc_W1, r2(enc_b1), mu1, sc1, r2(enc_be1))

    hi = lax.Precision.HIGHEST
    mh = hsum / N_EDGES
    cov = (hth / N_EDGES
           - jnp.dot(mh.T, mh, precision=hi))
    mu2 = jnp.dot(mh, enc_W2, precision=hi) + enc_b2[None, :]
    var2 = jnp.sum(enc_W2 * jnp.dot(cov, enc_W2, precision=hi),
                   axis=0, keepdims=True)
    s2 = enc_g2[None, :] * lax.rsqrt(var2 + EPS)
    a_vec = (enc_b2[None, :] - mu2) * s2 + enc_be2[None, :]
    w2s = enc_W2 * s2
    wcat = w2s.reshape(DIM, DIM, DIM).transpose(1, 0, 2).reshape(DIM, DIM * DIM)
    a_mat = a_vec.reshape(DIM, DIM)
    kk = jnp.arange(DIM * DIM, dtype=jnp.int32)
    r_mat = (jnp.arange(DIM, dtype=jnp.int32)[:, None] == kk[None, :] // DIM)
    r_mat = r_mat.astype(f32)
    s_mat = (kk[:, None] % DIM ==
             jnp.arange(DIM, dtype=jnp.int32)[None, :]).astype(f32)

    src_p = jnp.pad(src, (0, E_PAD - N_EDGES)).reshape(NW, NCH_E, CH)
    dst_p = jnp.pad(dst, (0, E_PAD - N_EDGES),
                    constant_values=N_NODES).reshape(NW, NCH_E, CH)
    hep = jnp.pad(he, ((0, E_PAD - N_EDGES), (0, 0)))
    zeros_acc = jnp.zeros((N_ACC, DIM), f32)
    zeros_acc16 = jnp.zeros((N_ACC, 16), f32)
    onesp = jnp.zeros((E_PAD, 16), f32).at[:N_EDGES].set(1.0)

    cnt_part = _sc_scatter_add(onesp.reshape(NW, NCH_E * CH, 16), dst_p,
                               zeros_acc16, NCH_E, CH, 16, nround=4)
    cnt_part = cnt_part[:, :N_NODES, :]

    ET3 = 4096
    T3 = E_PAD // ET3
    h = out0
    out = out0
    wihT = gru_Wih.T
    whhT = gru_Whh.T
    for _ in range(STEPS):
        xj0 = _sc_gather(out, src_p, NCH_E, CH, DIM, nround=2).reshape(E_PAD, DIM)
        msgp = pl.pallas_call(
            _msg_body,
            grid=(T3,),
            in_specs=[pl.BlockSpec((ET3, DIM), lambda i: (i, 0)),
                      pl.BlockSpec((ET3, DIM), lambda i: (i, 0)),
                      pl.BlockSpec((DIM, DIM * DIM), lambda i: (0, 0)),
                      pl.BlockSpec((DIM, DIM), lambda i: (0, 0)),
                      pl.BlockSpec((DIM, DIM * DIM), lambda i: (0, 0)),
                      pl.BlockSpec((DIM * DIM, DIM), lambda i: (0, 0)),
                      pl.BlockSpec((DIM, DIM), lambda i: (0, 0)),
                      pl.BlockSpec((DIM, DIM), lambda i: (0, 0)),
                      pl.BlockSpec((1, DIM), lambda i: (0, 0))],
            out_specs=pl.BlockSpec((ET3, DIM), lambda i: (i, 0)),
            out_shape=jax.ShapeDtypeStruct((E_PAD, DIM), f32),
        )(hep, xj0, wcat, a_mat, r_mat, s_mat, lin0_W, lin1_W, r2(lin1_b))
        part = _sc_scatter_add(msgp.reshape(NW, NCH_E * CH, DIM), dst_p,
                               zeros_acc, NCH_E, CH, DIM, nround=4)
        h, out = pl.pallas_call(
            _gru_body,
            out_shape=[jax.ShapeDtypeStruct((N_NODES, DIM), f32),
                       jax.ShapeDtypeStruct((N_NODES, DIM), f32)],
        )(part[:, :N_NODES, :], cnt_part, r2(conv_bias), h, wihT, whhT,
          r2(gru_bih), r2(gru_bhh), r2(ln_g), r2(ln_b))

    table = pl.pallas_call(
        _s2s_body,
        out_shape=jax.ShapeDtypeStruct((N_NODES, 3 * DIM), f32),
    )(out, batch.astype(jnp.int32).reshape(N_NODES, 1), s2s_Wih.T, s2s_Whh.T,
      r2(s2s_bih), r2(s2s_bhh))

    atom0 = target_index[0].astype(jnp.int32)
    atom1 = target_index[1].astype(jnp.int32)
    pair_idx = jnp.concatenate([atom0, atom1]).reshape(NW, 1, 2 * NPAIR // NW)
    g2 = _sc_gather(table, pair_idx, 1, 2 * NPAIR // NW, 3 * DIM)
    g2 = g2.reshape(2 * NPAIR, 3 * DIM)

    res = pl.pallas_call(
        _head_body,
        out_shape=jax.ShapeDtypeStruct((NPAIR, 1), f32),
    )(g2, target_class.astype(jnp.int32).reshape(NPAIR, 1),
      pr_W1, r2(pr_b1), r2(pr_g1), r2(pr_be1),
      pr_W2, r2(pr_b2), r2(pr_g2), r2(pr_be2), pr_W3, r2(pr_b3))
    return res.reshape(NPAIR)


# speedup vs baseline: 2.5987x; 1.0366x over previous
import functools

import jax
import jax.numpy as jnp
from jax import lax
from jax.experimental import pallas as pl
from jax.experimental.pallas import tpu as pltpu
from jax.experimental.pallas import tpu_sc as plsc

N_NODES = 10000
N_EDGES = 160000
NF = 128
EF = 16
DIM = 32
NG = 64
NPAIR = 1024
STEPS = 3
NOUT = 8
EPS = 1e-5

NW = 32
CH = 128
E_PAD = 163840
NCH_E = E_PAD // (NW * CH)
N_ACC = 10016


def _pre_body(x_ref, w1_ref, b1_ref, g1_ref, be1_ref, w2_ref, b2_ref,
              g2_ref, be2_ref, out_ref):
    x = x_ref[...]
    y = jnp.dot(x, w1_ref[...], preferred_element_type=jnp.float32) + b1_ref[...]
    mu = jnp.mean(y, axis=0, keepdims=True)
    var = jnp.mean((y - mu) ** 2, axis=0, keepdims=True)
    y = (y - mu) * (g1_ref[...] * lax.rsqrt(var + EPS)) + be1_ref[...]
    y = jnp.maximum(y, 0.0)
    z = jnp.dot(y, w2_ref[...], preferred_element_type=jnp.float32) + b2_ref[...]
    mu2 = jnp.mean(z, axis=0, keepdims=True)
    var2 = jnp.mean((z - mu2) ** 2, axis=0, keepdims=True)
    z = (z - mu2) * (g2_ref[...] * lax.rsqrt(var2 + EPS)) + be2_ref[...]
    out_ref[...] = jnp.maximum(z, 0.0)


def _enc_stats_body(ea_ref, w1_ref, b1_ref, sum_ref, sq_ref):
    @pl.when(pl.program_id(0) == 0)
    def _():
        sum_ref[...] = jnp.zeros_like(sum_ref)
        sq_ref[...] = jnp.zeros_like(sq_ref)
    y = jnp.dot(ea_ref[...], w1_ref[...], preferred_element_type=jnp.float32) + b1_ref[...]
    sum_ref[...] += jnp.sum(y, axis=0, keepdims=True)
    sq_ref[...] += jnp.sum(y * y, axis=0, keepdims=True)


def _enc_he_body(ea_ref, w1_ref, b1_ref, mu_ref, sc_ref, be_ref,
                 he_ref, hsum_ref, hh_ref):
    @pl.when(pl.program_id(0) == 0)
    def _():
        hsum_ref[...] = jnp.zeros_like(hsum_ref)
        hh_ref[...] = jnp.zeros_like(hh_ref)
    y = jnp.dot(ea_ref[...], w1_ref[...], preferred_element_type=jnp.float32) + b1_ref[...]
    he = jnp.maximum((y - mu_ref[...]) * sc_ref[...] + be_ref[...], 0.0)
    he_ref[...] = he
    hsum_ref[...] += jnp.sum(he, axis=0, keepdims=True)
    hh_ref[...] += lax.dot_general(he, he, (((0,), (0,)), ((), ())),
                                   preferred_element_type=jnp.float32,
                                   precision=lax.Precision.HIGHEST)


def _msg_body(he_ref, x0_ref, wcat_ref, amat_ref, rmat_ref, smat_ref,
              lin0_ref, lin1_ref, lin1b_ref, msg_ref):
    x0 = x0_ref[...]
    he = he_ref[...]
    u = jnp.dot(x0, wcat_ref[...], preferred_element_type=jnp.float32)
    hrep = jnp.dot(he, rmat_ref[...], preferred_element_type=jnp.float32)
    x1 = jnp.dot(hrep * u, smat_ref[...], preferred_element_type=jnp.float32)
    x1 = x1 + jnp.dot(x0, amat_ref[...], preferred_element_type=jnp.float32)
    coeff = jax.nn.sigmoid(
        jnp.dot(x0, lin0_ref[...], preferred_element_type=jnp.float32)
        + jnp.dot(x1, lin1_ref[...], preferred_element_type=jnp.float32)
        + lin1b_ref[...])
    msg_ref[...] = x1 + coeff * (x0 - x1)


def _gru_body(part_ref, cnt_ref, cb_ref, h_ref, wih_ref, whh_ref, bih_ref,
              bhh_ref, lng_ref, lnb_ref, hout_ref, out_ref):
    inv = 1.0 / jnp.maximum(cnt_ref[...][:, :1], 1.0)
    agg = part_ref[...] * inv + cb_ref[...]
    m = jnp.maximum(agg, 0.0)
    h = h_ref[...]
    gi = jnp.dot(m, wih_ref[...], preferred_element_type=jnp.float32) + bih_ref[...]
    gh = jnp.dot(h, whh_ref[...], preferred_element_type=jnp.float32) + bhh_ref[...]
    r = jax.nn.sigmoid(gi[:, :DIM] + gh[:, :DIM])
    z = jax.nn.sigmoid(gi[:, DIM:2 * DIM] + gh[:, DIM:2 * DIM])
    n = jnp.tanh(gi[:, 2 * DIM:] + r * gh[:, 2 * DIM:])
    h2 = (1.0 - z) * n + z * h
    hout_ref[...] = h2
    mu = jnp.mean(h2, axis=-1, keepdims=True)
    var = jnp.mean((h2 - mu) ** 2, axis=-1, keepdims=True)
    out_ref[...] = (h2 - mu) * lax.rsqrt(var + EPS) * lng_ref[...] + lnb_ref[...]


def _s2s_body(out_ref, batch_ref, wih_ref, whh_ref, bih_ref, bhh_ref,
              tab_ref):
    out = out_ref[...]
    b = batch_ref[...]
    giota = lax.broadcasted_iota(jnp.int32, (N_NODES, NG), 1)
    maskb = giota == b
    maskt = maskb.astype(jnp.float32)
    q = jnp.zeros((NG, 2 * DIM), jnp.float32)
    hh = jnp.zeros((NG, DIM), jnp.float32)
    cc = jnp.zeros((NG, DIM), jnp.float32)
    for _ in range(STEPS):
        g = (jnp.dot(q, wih_ref[...], preferred_element_type=jnp.float32)
             + bih_ref[...]
             + jnp.dot(hh, whh_ref[...], preferred_element_type=jnp.float32)
             + bhh_ref[...])
        i = jax.nn.sigmoid(g[:, :DIM])
        f = jax.nn.sigmoid(g[:, DIM:2 * DIM])
        gg = jnp.tanh(g[:, 2 * DIM:3 * DIM])
        o = jax.nn.sigmoid(g[:, 3 * DIM:])
        cc = f * cc + i * gg
        hh = o * jnp.tanh(cc)
        hb = jnp.dot(maskt, hh, preferred_element_type=jnp.float32)
        e = jnp.sum(out * hb, axis=-1, keepdims=True)
        em = jnp.max(jnp.where(maskb, e, -1e30), axis=0, keepdims=True)
        emb = jnp.sum(maskt * em, axis=-1, keepdims=True)
        a = jnp.exp(e - emb)
        asum = jnp.sum(maskt * a, axis=0, keepdims=True)
        asb = jnp.sum(maskt * asum, axis=-1, keepdims=True)
        an = a / (asb + 1e-16)
        r = lax.dot_general(maskt, an * out, (((0,), (0,)), ((), ())),
                            preferred_element_type=jnp.float32)
        q = jnp.concatenate([hh, r], axis=-1)
    tab_ref[:, :DIM] = out
    tab_ref[:, DIM:] = jnp.dot(maskt, q, preferred_element_type=jnp.float32)


def _head_body(g2_ref, tc_ref, w1_ref, b1_ref, g1_ref, be1_ref, w2_ref,
               b2_ref, g2w_ref, be2_ref, w3_ref, b3_ref, res_ref):
    node0 = g2_ref[:NPAIR, :DIM]
    s2s0 = g2_ref[:NPAIR, DIM:]
    node1 = g2_ref[NPAIR:, :DIM]
    feat = jnp.concatenate([node0, node1, s2s0], axis=-1)

    def ln_relu(v, g, be):
        mu = jnp.mean(v, axis=-1, keepdims=True)
        var = jnp.mean((v - mu) ** 2, axis=-1, keepdims=True)
        return jnp.maximum((v - mu) * lax.rsqrt(var + EPS) * g + be, 0.0)

    z = ln_relu(jnp.dot(feat, w1_ref[...], preferred_element_type=jnp.float32)
                + b1_ref[...], g1_ref[...], be1_ref[...])
    z = ln_relu(jnp.dot(z, w2_ref[...], preferred_element_type=jnp.float32)
                + b2_ref[...], g2w_ref[...], be2_ref[...])
    pred = jnp.dot(z, w3_ref[...], preferred_element_type=jnp.float32) + b3_ref[...]
    sel = lax.broadcasted_iota(jnp.int32, (NPAIR, NOUT), 1) == tc_ref[...]
    res_ref[...] = jnp.sum(jnp.where(sel, pred, 0.0), axis=-1, keepdims=True)


def _sc_gather(table, idxp, nch, ch, d, nround=1):
    cpr = nch // nround
    mesh = plsc.VectorSubcoreMesh(core_axis_name="c", subcore_axis_name="s")

    @functools.partial(
        pl.kernel, mesh=mesh,
        out_type=jax.ShapeDtypeStruct((NW, nch * ch, d), jnp.float32),
        compiler_params=pltpu.CompilerParams(use_tc_tiling_on_sc=False),
        scratch_types=[
            pltpu.VMEM((nch, ch), jnp.int32),
            pltpu.VMEM((cpr * ch, d), jnp.float32),
            pltpu.SemaphoreType.DMA,
        ],
    )
    def k(table_hbm, idx_hbm, out_hbm, idx_v, rows_v, sem):
        c = lax.axis_index("c")
        s = lax.axis_index("s")
        w = s * 2 + c
        pltpu.sync_copy(idx_hbm.at[w], idx_v)
        for r in range(nround):
            copies = []
            for j in range(cpr):
                copies.append(pltpu.async_copy(
                    table_hbm.at[idx_v.at[r * cpr + j]],
                    rows_v.at[pl.ds(j * ch, ch)], sem))
            for cp in copies:
                cp.wait()
            pltpu.sync_copy(rows_v,
                            out_hbm.at[w].at[pl.ds(r * cpr * ch, cpr * ch)])

    return k(table, idxp)


def _sc_scatter_add(msgp, idxp, zeros_acc, nch, ch, d, nround=1):
    cpr = nch // nround
    mesh = plsc.VectorSubcoreMesh(core_axis_name="c", subcore_axis_name="s")
    zrows = N_ACC // 16

    @functools.partial(
        pl.kernel, mesh=mesh,
        out_type=jax.ShapeDtypeStruct((N_ACC, d), jnp.float32),
        compiler_params=pltpu.CompilerParams(use_tc_tiling_on_sc=False),
        scratch_types=[
            pltpu.VMEM((2 * nch, ch), jnp.int32),
            pltpu.VMEM((2, cpr * ch, d), jnp.float32),
            pltpu.VMEM_SHARED((N_ACC, d), jnp.float32),
            pltpu.SemaphoreType.DMA,
        ],
    )
    def k(msg_hbm, idx_hbm, z_hbm, out_hbm, idx_v, msg_v, acc_sh, sem):
        c = lax.axis_index("c")
        s = lax.axis_index("s")

        @pl.when(c == 0)
        def _():
            pltpu.sync_copy(z_hbm.at[pl.ds(s * zrows, zrows)],
                            acc_sh.at[pl.ds(s * zrows, zrows)])
            plsc.subcore_barrier()
            pltpu.sync_copy(idx_hbm.at[2 * s], idx_v.at[pl.ds(0, nch)])
            pltpu.sync_copy(idx_hbm.at[2 * s + 1], idx_v.at[pl.ds(nch, nch)])
            pltpu.sync_copy(msg_hbm.at[2 * s].at[pl.ds(0, cpr * ch)],
                            msg_v.at[0])
            for rr in range(2 * nround):
                half, r = divmod(rr, nround)
                adds = []
                for j in range(cpr):
                    adds.append(pltpu.async_copy(
                        msg_v.at[rr % 2].at[pl.ds(j * ch, ch)],
                        acc_sh.at[idx_v.at[half * nch + r * cpr + j]],
                        sem, add=True))
                if rr + 1 < 2 * nround:
                    h2, r2 = divmod(rr + 1, nround)
                    pltpu.sync_copy(
                        msg_hbm.at[2 * s + h2].at[pl.ds(r2 * cpr * ch,
                                                        cpr * ch)],
                        msg_v.at[(rr + 1) % 2])
                for cp in adds:
                    cp.wait()
            plsc.subcore_barrier()
            pltpu.sync_copy(acc_sh.at[pl.ds(s * zrows, zrows)],
                            out_hbm.at[pl.ds(s * zrows, zrows)])

    return k(msgp, idxp, zeros_acc)


def _sc_count(idxp, zeros_acc, nch, ch):
    d = 16
    mesh = plsc.VectorSubcoreMesh(core_axis_name="c", subcore_axis_name="s")
    zrows = N_ACC // 16

    @functools.partial(
        pl.kernel, mesh=mesh,
        out_type=jax.ShapeDtypeStruct((N_ACC, d), jnp.float32),
        compiler_params=pltpu.CompilerParams(use_tc_tiling_on_sc=False),
        scratch_types=[
            pltpu.VMEM((2 * nch, ch), jnp.int32),
            pltpu.VMEM((ch, d), jnp.float32),
            pltpu.VMEM_SHARED((N_ACC, d), jnp.float32),
            pltpu.SemaphoreType.DMA,
        ],
    )
    def k(idx_hbm, z_hbm, out_hbm, idx_v, ones_v, acc_sh, sem):
        c = lax.axis_index("c")
        s = lax.axis_index("s")

        @pl.when(c == 0)
        def _():
            pltpu.sync_copy(z_hbm.at[pl.ds(s * zrows, zrows)],
                            acc_sh.at[pl.ds(s * zrows, zrows)])

            def fill(i, _):
                ones_v[i, :] = jnp.ones((d,), jnp.float32)
                return 0

            lax.fori_loop(0, ch, fill, 0)
            plsc.subcore_barrier()
            pltpu.sync_copy(idx_hbm.at[2 * s], idx_v.at[pl.ds(0, nch)])
            pltpu.sync_copy(idx_hbm.at[2 * s + 1], idx_v.at[pl.ds(nch, nch)])
            for j in range(2 * nch):
                pltpu.sync_copy(ones_v, acc_sh.at[idx_v.at[j]], add=True)
            plsc.subcore_barrier()
            pltpu.sync_copy(acc_sh.at[pl.ds(s * zrows, zrows)],
                            out_hbm.at[pl.ds(s * zrows, zrows)])

    return k(idxp, zeros_acc)


def kernel(x, edge_attr, edge_index, target_index, batch, target_class, pre_W1, pre_b1, pre_g1, pre_be1, pre_W2, pre_b2, pre_g2, pre_be2, enc_W1, enc_b1, enc_g1, enc_be1, enc_W2, enc_b2, enc_g2, enc_be2, lin0_W, lin1_W, lin1_b, conv_bias, gru_Wih, gru_Whh, gru_bih, gru_bhh, ln_g, ln_b, s2s_Wih, s2s_Whh, s2s_bih, s2s_bhh, pr_W1, pr_b1, pr_g1, pr_be1, pr_W2, pr_b2, pr_g2, pr_be2, pr_W3, pr_b3):
    f32 = jnp.float32
    r2 = lambda v: v.reshape(1, -1).astype(f32)
    src = edge_index[0].astype(jnp.int32)
    dst = edge_index[1].astype(jnp.int32)

    out0 = pl.pallas_call(
        _pre_body,
        out_shape=jax.ShapeDtypeStruct((N_NODES, DIM), f32),
    )(x, pre_W1, r2(pre_b1), r2(pre_g1), r2(pre_be1),
      pre_W2, r2(pre_b2), r2(pre_g2), r2(pre_be2))

    ET2 = 8000
    T2 = N_EDGES // ET2
    sum1, sq1 = pl.pallas_call(
        _enc_stats_body,
        grid=(T2,),
        in_specs=[pl.BlockSpec((ET2, EF), lambda i: (i, 0)),
                  pl.BlockSpec((EF, DIM), lambda i: (0, 0)),
                  pl.BlockSpec((1, DIM), lambda i: (0, 0))],
        out_specs=[pl.BlockSpec((1, DIM), lambda i: (0, 0)),
                   pl.BlockSpec((1, DIM), lambda i: (0, 0))],
        out_shape=[jax.ShapeDtypeStruct((1, DIM), f32),
                   jax.ShapeDtypeStruct((1, DIM), f32)],
    )(edge_attr, enc_W1, r2(enc_b1))
    mu1 = sum1 / N_EDGES
    var1 = sq1 / N_EDGES - mu1 * mu1
    sc1 = r2(enc_g1) * lax.rsqrt(var1 + EPS)

    he, hsum, hth = pl.pallas_call(
        _enc_he_body,
        grid=(T2,),
        in_specs=[pl.BlockSpec((ET2, EF), lambda i: (i, 0)),
                  pl.BlockSpec((EF, DIM), lambda i: (0, 0)),
                  pl.BlockSpec((1, DIM), lambda i: (0, 0)),
                  pl.BlockSpec((1, DIM), lambda i: (0, 0)),
                  pl.BlockSpec((1, DIM), lambda i: (0, 0)),
                  pl.BlockSpec((1, DIM), lambda i: (0, 0))],
        out_specs=[pl.BlockSpec((ET2, DIM), lambda i: (i, 0)),
                   pl.BlockSpec((1, DIM), lambda i: (0, 0)),
                   pl.BlockSpec((DIM, DIM), lambda i: (0, 0))],
        out_shape=[jax.ShapeDtypeStruct((N_EDGES, DIM), f32),
                   jax.ShapeDtypeStruct((1, DIM), f32),
                   jax.ShapeDtypeStruct((DIM, DIM), f32)],
    )(edge_attr, enc_W1, r2(enc_b1), mu1, sc1, r2(enc_be1))

    hi = lax.Precision.HIGHEST
    mh = hsum / N_EDGES
    cov = (hth / N_EDGES
           - jnp.dot(mh.T, mh, precision=hi))
    mu2 = jnp.dot(mh, enc_W2, precision=hi) + enc_b2[None, :]
    var2 = jnp.sum(enc_W2 * jnp.dot(cov, enc_W2, precision=hi),
                   axis=0, keepdims=True)
    s2 = enc_g2[None, :] * lax.rsqrt(var2 + EPS)
    a_vec = (enc_b2[None, :] - mu2) * s2 + enc_be2[None, :]
    w2s = enc_W2 * s2
    wcat = w2s.reshape(DIM, DIM, DIM).transpose(1, 0, 2).reshape(DIM, DIM * DIM)
    a_mat = a_vec.reshape(DIM, DIM)
    kk = jnp.arange(DIM * DIM, dtype=jnp.int32)
    r_mat = (jnp.arange(DIM, dtype=jnp.int32)[:, None] == kk[None, :] // DIM)
    r_mat = r_mat.astype(f32)
    s_mat = (kk[:, None] % DIM ==
             jnp.arange(DIM, dtype=jnp.int32)[None, :]).astype(f32)

    src_p = jnp.pad(src, (0, E_PAD - N_EDGES)).reshape(NW, NCH_E, CH)
    dst_p = jnp.pad(dst, (0, E_PAD - N_EDGES),
                    constant_values=N_NODES).reshape(NW, NCH_E, CH)
    zeros_acc = jnp.zeros((N_ACC, DIM), f32)
    zeros_acc16 = jnp.zeros((N_ACC, 16), f32)

    cnt_acc = _sc_count(dst_p, zeros_acc16, NCH_E, CH)[:N_NODES, :]

    ET3 = 4096
    T3 = E_PAD // ET3
    h = out0
    out = out0
    wihT = gru_Wih.T
    whhT = gru_Whh.T
    for _ in range(STEPS):
        xj0 = _sc_gather(out, src_p, NCH_E, CH, DIM, nround=2).reshape(E_PAD, DIM)
        msgp = pl.pallas_call(
            _msg_body,
            grid=(T3,),
            in_specs=[pl.BlockSpec((ET3, DIM), lambda i: (i, 0)),
                      pl.BlockSpec((ET3, DIM), lambda i: (i, 0)),
                      pl.BlockSpec((DIM, DIM * DIM), lambda i: (0, 0)),
                      pl.BlockSpec((DIM, DIM), lambda i: (0, 0)),
                      pl.BlockSpec((DIM, DIM * DIM), lambda i: (0, 0)),
                      pl.BlockSpec((DIM * DIM, DIM), lambda i: (0, 0)),
                      pl.BlockSpec((DIM, DIM), lambda i: (0, 0)),
                      pl.BlockSpec((DIM, DIM), lambda i: (0, 0)),
                      pl.BlockSpec((1, DIM), lambda i: (0, 0))],
            out_specs=pl.BlockSpec((ET3, DIM), lambda i: (i, 0)),
            out_shape=jax.ShapeDtypeStruct((E_PAD, DIM), f32),
        )(he, xj0, wcat, a_mat, r_mat, s_mat, lin0_W, lin1_W, r2(lin1_b))
        part = _sc_scatter_add(msgp.reshape(NW, NCH_E * CH, DIM), dst_p,
                               zeros_acc, NCH_E, CH, DIM, nround=4)
        h, out = pl.pallas_call(
            _gru_body,
            out_shape=[jax.ShapeDtypeStruct((N_NODES, DIM), f32),
                       jax.ShapeDtypeStruct((N_NODES, DIM), f32)],
        )(part[:N_NODES, :], cnt_acc, r2(conv_bias), h, wihT, whhT,
          r2(gru_bih), r2(gru_bhh), r2(ln_g), r2(ln_b))

    table = pl.pallas_call(
        _s2s_body,
        out_shape=jax.ShapeDtypeStruct((N_NODES, 3 * DIM), f32),
    )(out, batch.astype(jnp.int32).reshape(N_NODES, 1), s2s_Wih.T, s2s_Whh.T,
      r2(s2s_bih), r2(s2s_bhh))

    atom0 = target_index[0].astype(jnp.int32)
    atom1 = target_index[1].astype(jnp.int32)
    pair_idx = jnp.concatenate([atom0, atom1]).reshape(NW, 1, 2 * NPAIR // NW)
    g2 = _sc_gather(table, pair_idx, 1, 2 * NPAIR // NW, 3 * DIM)
    g2 = g2.reshape(2 * NPAIR, 3 * DIM)

    res = pl.pallas_call(
        _head_body,
        out_shape=jax.ShapeDtypeStruct((NPAIR, 1), f32),
    )(g2, target_class.astype(jnp.int32).reshape(NPAIR, 1),
      pr_W1, r2(pr_b1), r2(pr_g1), r2(pr_be1),
      pr_W2, r2(pr_b2), r2(pr_g2), r2(pr_be2), pr_W3, r2(pr_b3))
    return res.reshape(NPAIR)


# speedup vs baseline: 2.7917x; 1.0743x over previous
import functools

import jax
import jax.numpy as jnp
from jax import lax
from jax.experimental import pallas as pl
from jax.experimental.pallas import tpu as pltpu
from jax.experimental.pallas import tpu_sc as plsc

N_NODES = 10000
N_EDGES = 160000
NF = 128
EF = 16
DIM = 32
NG = 64
NPAIR = 1024
STEPS = 3
NOUT = 8
EPS = 1e-5

NW = 32
CH = 128
E_PAD = 163840
NCH_E = E_PAD // (NW * CH)
N_ACC = 10016


def _pre_body(x_ref, w1_ref, b1_ref, g1_ref, be1_ref, w2_ref, b2_ref,
              g2_ref, be2_ref, out_ref):
    x = x_ref[...]
    y = jnp.dot(x, w1_ref[...], preferred_element_type=jnp.float32) + b1_ref[...]
    mu = jnp.mean(y, axis=0, keepdims=True)
    var = jnp.mean((y - mu) ** 2, axis=0, keepdims=True)
    y = (y - mu) * (g1_ref[...] * lax.rsqrt(var + EPS)) + be1_ref[...]
    y = jnp.maximum(y, 0.0)
    z = jnp.dot(y, w2_ref[...], preferred_element_type=jnp.float32) + b2_ref[...]
    mu2 = jnp.mean(z, axis=0, keepdims=True)
    var2 = jnp.mean((z - mu2) ** 2, axis=0, keepdims=True)
    z = (z - mu2) * (g2_ref[...] * lax.rsqrt(var2 + EPS)) + be2_ref[...]
    out_ref[...] = jnp.maximum(z, 0.0)


def _enc_stats_body(ea_ref, w1_ref, b1_ref, sum_ref, sq_ref):
    @pl.when(pl.program_id(0) == 0)
    def _():
        sum_ref[...] = jnp.zeros_like(sum_ref)
        sq_ref[...] = jnp.zeros_like(sq_ref)
    y = jnp.dot(ea_ref[...], w1_ref[...], preferred_element_type=jnp.float32) + b1_ref[...]
    sum_ref[...] += jnp.sum(y, axis=0, keepdims=True)
    sq_ref[...] += jnp.sum(y * y, axis=0, keepdims=True)


def _enc_he_body(ea_ref, w1_ref, b1_ref, mu_ref, sc_ref, be_ref,
                 he_ref, hsum_ref, hh_ref):
    @pl.when(pl.program_id(0) == 0)
    def _():
        hsum_ref[...] = jnp.zeros_like(hsum_ref)
        hh_ref[...] = jnp.zeros_like(hh_ref)
    y = jnp.dot(ea_ref[...], w1_ref[...], preferred_element_type=jnp.float32) + b1_ref[...]
    he = jnp.maximum((y - mu_ref[...]) * sc_ref[...] + be_ref[...], 0.0)
    he_ref[...] = he
    hsum_ref[...] += jnp.sum(he, axis=0, keepdims=True)
    hh_ref[...] += lax.dot_general(he, he, (((0,), (0,)), ((), ())),
                                   preferred_element_type=jnp.float32,
                                   precision=lax.Precision.HIGHEST)


def _msg_body(he_ref, x0_ref, wcat_ref, amat_ref, rmat_ref, smat_ref,
              lin0_ref, lin1_ref, lin1b_ref, msg_ref):
    x0 = x0_ref[...]
    he = he_ref[...]
    u = jnp.dot(x0, wcat_ref[...], preferred_element_type=jnp.float32)
    hrep = jnp.dot(he, rmat_ref[...], preferred_element_type=jnp.float32)
    x1 = jnp.dot(hrep * u, smat_ref[...], preferred_element_type=jnp.float32)
    x1 = x1 + jnp.dot(x0, amat_ref[...], preferred_element_type=jnp.float32)
    coeff = jax.nn.sigmoid(
        jnp.dot(x0, lin0_ref[...], preferred_element_type=jnp.float32)
        + jnp.dot(x1, lin1_ref[...], preferred_element_type=jnp.float32)
        + lin1b_ref[...])
    msg_ref[...] = x1 + coeff * (x0 - x1)


def _gru_body(part_ref, cnt_ref, cb_ref, h_ref, wih_ref, whh_ref, bih_ref,
              bhh_ref, lng_ref, lnb_ref, hout_ref, out_ref):
    inv = 1.0 / jnp.maximum(cnt_ref[...][:, :1], 1.0)
    agg = part_ref[...] * inv + cb_ref[...]
    m = jnp.maximum(agg, 0.0)
    h = h_ref[...]
    gi = jnp.dot(m, wih_ref[...], preferred_element_type=jnp.float32) + bih_ref[...]
    gh = jnp.dot(h, whh_ref[...], preferred_element_type=jnp.float32) + bhh_ref[...]
    r = jax.nn.sigmoid(gi[:, :DIM] + gh[:, :DIM])
    z = jax.nn.sigmoid(gi[:, DIM:2 * DIM] + gh[:, DIM:2 * DIM])
    n = jnp.tanh(gi[:, 2 * DIM:] + r * gh[:, 2 * DIM:])
    h2 = (1.0 - z) * n + z * h
    hout_ref[...] = h2
    mu = jnp.mean(h2, axis=-1, keepdims=True)
    var = jnp.mean((h2 - mu) ** 2, axis=-1, keepdims=True)
    out_ref[...] = (h2 - mu) * lax.rsqrt(var + EPS) * lng_ref[...] + lnb_ref[...]


def _s2s_body(out_ref, batch_ref, wih_ref, whh_ref, bih_ref, bhh_ref,
              tab_ref):
    out = out_ref[...]
    b = batch_ref[...]
    giota = lax.broadcasted_iota(jnp.int32, (N_NODES, NG), 1)
    maskb = giota == b
    maskt = maskb.astype(jnp.float32)
    q = jnp.zeros((NG, 2 * DIM), jnp.float32)
    hh = jnp.zeros((NG, DIM), jnp.float32)
    cc = jnp.zeros((NG, DIM), jnp.float32)
    for _ in range(STEPS):
        g = (jnp.dot(q, wih_ref[...], preferred_element_type=jnp.float32)
             + bih_ref[...]
             + jnp.dot(hh, whh_ref[...], preferred_element_type=jnp.float32)
             + bhh_ref[...])
        i = jax.nn.sigmoid(g[:, :DIM])
        f = jax.nn.sigmoid(g[:, DIM:2 * DIM])
        gg = jnp.tanh(g[:, 2 * DIM:3 * DIM])
        o = jax.nn.sigmoid(g[:, 3 * DIM:])
        cc = f * cc + i * gg
        hh = o * jnp.tanh(cc)
        hb = jnp.dot(maskt, hh, preferred_element_type=jnp.float32)
        e = jnp.sum(out * hb, axis=-1, keepdims=True)
        em = jnp.max(jnp.where(maskb, e, -1e30), axis=0, keepdims=True)
        emb = jnp.sum(maskt * em, axis=-1, keepdims=True)
        a = jnp.exp(e - emb)
        asum = jnp.sum(maskt * a, axis=0, keepdims=True)
        asb = jnp.sum(maskt * asum, axis=-1, keepdims=True)
        an = a / (asb + 1e-16)
        r = lax.dot_general(maskt, an * out, (((0,), (0,)), ((), ())),
                            preferred_element_type=jnp.float32)
        q = jnp.concatenate([hh, r], axis=-1)
    tab_ref[:, :DIM] = out
    tab_ref[:, DIM:] = jnp.dot(maskt, q, preferred_element_type=jnp.float32)


def _head_body(g2_ref, tc_ref, w1_ref, b1_ref, g1_ref, be1_ref, w2_ref,
               b2_ref, g2w_ref, be2_ref, w3_ref, b3_ref, res_ref):
    node0 = g2_ref[:NPAIR, :DIM]
    s2s0 = g2_ref[:NPAIR, DIM:]
    node1 = g2_ref[NPAIR:, :DIM]
    feat = jnp.concatenate([node0, node1, s2s0], axis=-1)

    def ln_relu(v, g, be):
        mu = jnp.mean(v, axis=-1, keepdims=True)
        var = jnp.mean((v - mu) ** 2, axis=-1, keepdims=True)
        return jnp.maximum((v - mu) * lax.rsqrt(var + EPS) * g + be, 0.0)

    z = ln_relu(jnp.dot(feat, w1_ref[...], preferred_element_type=jnp.float32)
                + b1_ref[...], g1_ref[...], be1_ref[...])
    z = ln_relu(jnp.dot(z, w2_ref[...], preferred_element_type=jnp.float32)
                + b2_ref[...], g2w_ref[...], be2_ref[...])
    pred = jnp.dot(z, w3_ref[...], preferred_element_type=jnp.float32) + b3_ref[...]
    sel = lax.broadcasted_iota(jnp.int32, (NPAIR, NOUT), 1) == tc_ref[...]
    res_ref[...] = jnp.sum(jnp.where(sel, pred, 0.0), axis=-1, keepdims=True)


def _sc_gather(table, idxp, nch, ch, d, nround=1):
    cpr = nch // nround
    mesh = plsc.VectorSubcoreMesh(core_axis_name="c", subcore_axis_name="s")

    @functools.partial(
        pl.kernel, mesh=mesh,
        out_type=jax.ShapeDtypeStruct((NW, nch * ch, d), jnp.float32),
        compiler_params=pltpu.CompilerParams(use_tc_tiling_on_sc=False),
        scratch_types=[
            pltpu.VMEM((nch, ch), jnp.int32),
            pltpu.VMEM((cpr * ch, d), jnp.float32),
            pltpu.SemaphoreType.DMA,
        ],
    )
    def k(table_hbm, idx_hbm, out_hbm, idx_v, rows_v, sem):
        c = lax.axis_index("c")
        s = lax.axis_index("s")
        w = s * 2 + c
        pltpu.sync_copy(idx_hbm.at[w], idx_v)
        for r in range(nround):
            copies = []
            for j in range(cpr):
                copies.append(pltpu.async_copy(
                    table_hbm.at[idx_v.at[r * cpr + j]],
                    rows_v.at[pl.ds(j * ch, ch)], sem))
            for cp in copies:
                cp.wait()
            pltpu.sync_copy(rows_v,
                            out_hbm.at[w].at[pl.ds(r * cpr * ch, cpr * ch)])

    return k(table, idxp)


def _sc_scatter_add(msgp, idxp, zeros_acc, nch, ch, d, nround=1):
    cpr = nch // nround
    mesh = plsc.VectorSubcoreMesh(core_axis_name="c", subcore_axis_name="s")
    zrows = N_ACC // 16

    @functools.partial(
        pl.kernel, mesh=mesh,
        out_type=jax.ShapeDtypeStruct((N_ACC, d), jnp.float32),
        compiler_params=pltpu.CompilerParams(use_tc_tiling_on_sc=False),
        scratch_types=[
            pltpu.VMEM((2 * nch, ch), jnp.int32),
            pltpu.VMEM((2, cpr * ch, d), jnp.float32),
            pltpu.VMEM_SHARED((N_ACC, d), jnp.float32),
            pltpu.SemaphoreType.DMA,
        ],
    )
    def k(msg_hbm, idx_hbm, z_hbm, out_hbm, idx_v, msg_v, acc_sh, sem):
        c = lax.axis_index("c")
        s = lax.axis_index("s")

        @pl.when(c == 0)
        def _():
            pltpu.sync_copy(z_hbm.at[pl.ds(s * zrows, zrows)],
                            acc_sh.at[pl.ds(s * zrows, zrows)])
            plsc.subcore_barrier()
            pltpu.sync_copy(idx_hbm.at[2 * s], idx_v.at[pl.ds(0, nch)])
            pltpu.sync_copy(idx_hbm.at[2 * s + 1], idx_v.at[pl.ds(nch, nch)])
            pltpu.sync_copy(msg_hbm.at[2 * s].at[pl.ds(0, cpr * ch)],
                            msg_v.at[0])
            for rr in range(2 * nround):
                half, r = divmod(rr, nround)
                adds = []
                for j in range(cpr):
                    adds.append(pltpu.async_copy(
                        msg_v.at[rr % 2].at[pl.ds(j * ch, ch)],
                        acc_sh.at[idx_v.at[half * nch + r * cpr + j]],
                        sem, add=True))
                if rr + 1 < 2 * nround:
                    h2, r2 = divmod(rr + 1, nround)
                    pltpu.sync_copy(
                        msg_hbm.at[2 * s + h2].at[pl.ds(r2 * cpr * ch,
                                                        cpr * ch)],
                        msg_v.at[(rr + 1) % 2])
                for cp in adds:
                    cp.wait()
            plsc.subcore_barrier()
            pltpu.sync_copy(acc_sh.at[pl.ds(s * zrows, zrows)],
                            out_hbm.at[pl.ds(s * zrows, zrows)])

    return k(msgp, idxp, zeros_acc)


def _sc_count(idxp, zeros_acc, nch, ch):
    d = 16
    mesh = plsc.VectorSubcoreMesh(core_axis_name="c", subcore_axis_name="s")
    zrows = N_ACC // 16

    @functools.partial(
        pl.kernel, mesh=mesh,
        out_type=jax.ShapeDtypeStruct((N_ACC, d), jnp.float32),
        compiler_params=pltpu.CompilerParams(use_tc_tiling_on_sc=False),
        scratch_types=[
            pltpu.VMEM((2 * nch, ch), jnp.int32),
            pltpu.VMEM((ch, d), jnp.float32),
            pltpu.VMEM_SHARED((N_ACC, d), jnp.float32),
            pltpu.SemaphoreType.DMA,
        ],
    )
    def k(idx_hbm, z_hbm, out_hbm, idx_v, ones_v, acc_sh, sem):
        c = lax.axis_index("c")
        s = lax.axis_index("s")

        @pl.when(c == 0)
        def _():
            pltpu.sync_copy(z_hbm.at[pl.ds(s * zrows, zrows)],
                            acc_sh.at[pl.ds(s * zrows, zrows)])

            def fill(i, _):
                ones_v[i, :] = jnp.ones((d,), jnp.float32)
                return 0

            lax.fori_loop(0, ch, fill, 0)
            plsc.subcore_barrier()
            pltpu.sync_copy(idx_hbm.at[2 * s], idx_v.at[pl.ds(0, nch)])
            pltpu.sync_copy(idx_hbm.at[2 * s + 1], idx_v.at[pl.ds(nch, nch)])
            for j in range(2 * nch):
                pltpu.sync_copy(ones_v, acc_sh.at[idx_v.at[j]], add=True)
            plsc.subcore_barrier()
            pltpu.sync_copy(acc_sh.at[pl.ds(s * zrows, zrows)],
                            out_hbm.at[pl.ds(s * zrows, zrows)])

    return k(idxp, zeros_acc)


def kernel(x, edge_attr, edge_index, target_index, batch, target_class, pre_W1, pre_b1, pre_g1, pre_be1, pre_W2, pre_b2, pre_g2, pre_be2, enc_W1, enc_b1, enc_g1, enc_be1, enc_W2, enc_b2, enc_g2, enc_be2, lin0_W, lin1_W, lin1_b, conv_bias, gru_Wih, gru_Whh, gru_bih, gru_bhh, ln_g, ln_b, s2s_Wih, s2s_Whh, s2s_bih, s2s_bhh, pr_W1, pr_b1, pr_g1, pr_be1, pr_W2, pr_b2, pr_g2, pr_be2, pr_W3, pr_b3):
    f32 = jnp.float32
    r2 = lambda v: v.reshape(1, -1).astype(f32)
    src = edge_index[0].astype(jnp.int32)
    dst = edge_index[1].astype(jnp.int32)

    out0 = pl.pallas_call(
        _pre_body,
        out_shape=jax.ShapeDtypeStruct((N_NODES, DIM), f32),
    )(x, pre_W1, r2(pre_b1), r2(pre_g1), r2(pre_be1),
      pre_W2, r2(pre_b2), r2(pre_g2), r2(pre_be2))

    ET2 = 8000
    T2 = N_EDGES // ET2
    sum1, sq1 = pl.pallas_call(
        _enc_stats_body,
        grid=(T2,),
        in_specs=[pl.BlockSpec((ET2, EF), lambda i: (i, 0)),
                  pl.BlockSpec((EF, DIM), lambda i: (0, 0)),
                  pl.BlockSpec((1, DIM), lambda i: (0, 0))],
        out_specs=[pl.BlockSpec((1, DIM), lambda i: (0, 0)),
                   pl.BlockSpec((1, DIM), lambda i: (0, 0))],
        out_shape=[jax.ShapeDtypeStruct((1, DIM), f32),
                   jax.ShapeDtypeStruct((1, DIM), f32)],
    )(edge_attr, enc_W1, r2(enc_b1))
    mu1 = sum1 / N_EDGES
    var1 = sq1 / N_EDGES - mu1 * mu1
    sc1 = r2(enc_g1) * lax.rsqrt(var1 + EPS)

    he, hsum, hth = pl.pallas_call(
        _enc_he_body,
        grid=(T2,),
        in_specs=[pl.BlockSpec((ET2, EF), lambda i: (i, 0)),
                  pl.BlockSpec((EF, DIM), lambda i: (0, 0)),
                  pl.BlockSpec((1, DIM), lambda i: (0, 0)),
                  pl.BlockSpec((1, DIM), lambda i: (0, 0)),
                  pl.BlockSpec((1, DIM), lambda i: (0, 0)),
                  pl.BlockSpec((1, DIM), lambda i: (0, 0))],
        out_specs=[pl.BlockSpec((ET2, DIM), lambda i: (i, 0)),
                   pl.BlockSpec((1, DIM), lambda i: (0, 0)),
                   pl.BlockSpec((DIM, DIM), lambda i: (0, 0))],
        out_shape=[jax.ShapeDtypeStruct((N_EDGES, DIM), f32),
                   jax.ShapeDtypeStruct((1, DIM), f32),
                   jax.ShapeDtypeStruct((DIM, DIM), f32)],
    )(edge_attr, enc_W1, r2(enc_b1), mu1, sc1, r2(enc_be1))

    hi = lax.Precision.HIGHEST
    mh = hsum / N_EDGES
    cov = (hth / N_EDGES
           - jnp.dot(mh.T, mh, precision=hi))
    mu2 = jnp.dot(mh, enc_W2, precision=hi) + enc_b2[None, :]
    var2 = jnp.sum(enc_W2 * jnp.dot(cov, enc_W2, precision=hi),
                   axis=0, keepdims=True)
    s2 = enc_g2[None, :] * lax.rsqrt(var2 + EPS)
    a_vec = (enc_b2[None, :] - mu2) * s2 + enc_be2[None, :]
    w2s = enc_W2 * s2
    wcat = w2s.reshape(DIM, DIM, DIM).transpose(1, 0, 2).reshape(DIM, DIM * DIM)
    a_mat = a_vec.reshape(DIM, DIM)
    kk = jnp.arange(DIM * DIM, dtype=jnp.int32)
    r_mat = (jnp.arange(DIM, dtype=jnp.int32)[:, None] == kk[None, :] // DIM)
    r_mat = r_mat.astype(f32)
    s_mat = (kk[:, None] % DIM ==
             jnp.arange(DIM, dtype=jnp.int32)[None, :]).astype(f32)
    eye4 = jnp.eye(4, dtype=f32)
    wcat4 = jnp.kron(eye4, wcat)
    rmat4 = jnp.kron(eye4, r_mat)
    smat4 = jnp.kron(eye4, s_mat)
    amat4 = jnp.kron(eye4, a_mat)
    lin0_4 = jnp.kron(eye4, lin0_W)
    lin1_4 = jnp.kron(eye4, lin1_W)
    lin1b4 = jnp.tile(lin1_b, 4)[None, :]

    src_p = jnp.pad(src, (0, E_PAD - N_EDGES)).reshape(NW, NCH_E, CH)
    dst_p = jnp.pad(dst, (0, E_PAD - N_EDGES),
                    constant_values=N_NODES).reshape(NW, NCH_E, CH)
    zeros_acc = jnp.zeros((N_ACC, DIM), f32)
    zeros_acc16 = jnp.zeros((N_ACC, 16), f32)

    cnt_acc = _sc_count(dst_p, zeros_acc16, NCH_E, CH)[:N_NODES, :]

    EP4 = E_PAD // 4
    ET3 = 512
    T3 = EP4 // ET3
    D4 = 4 * DIM
    he4 = jnp.pad(he, ((0, E_PAD - N_EDGES), (0, 0))).reshape(EP4, D4)
    h = out0
    out = out0
    wihT = gru_Wih.T
    whhT = gru_Whh.T
    for _ in range(STEPS):
        xj0 = _sc_gather(out, src_p, NCH_E, CH, DIM, nround=2).reshape(EP4, D4)
        msgp = pl.pallas_call(
            _msg_body,
            grid=(T3,),
            in_specs=[pl.BlockSpec((ET3, D4), lambda i: (i, 0)),
                      pl.BlockSpec((ET3, D4), lambda i: (i, 0)),
                      pl.BlockSpec((D4, 4 * DIM * DIM), lambda i: (0, 0)),
                      pl.BlockSpec((D4, D4), lambda i: (0, 0)),
                      pl.BlockSpec((D4, 4 * DIM * DIM), lambda i: (0, 0)),
                      pl.BlockSpec((4 * DIM * DIM, D4), lambda i: (0, 0)),
                      pl.BlockSpec((D4, D4), lambda i: (0, 0)),
                      pl.BlockSpec((D4, D4), lambda i: (0, 0)),
                      pl.BlockSpec((1, D4), lambda i: (0, 0))],
            out_specs=pl.BlockSpec((ET3, D4), lambda i: (i, 0)),
            out_shape=jax.ShapeDtypeStruct((EP4, D4), f32),
        )(he4, xj0, wcat4, amat4, rmat4, smat4, lin0_4, lin1_4, lin1b4)
        part = _sc_scatter_add(msgp.reshape(NW, NCH_E * CH, DIM), dst_p,
                               zeros_acc, NCH_E, CH, DIM, nround=4)
        h, out = pl.pallas_call(
            _gru_body,
            out_shape=[jax.ShapeDtypeStruct((N_NODES, DIM), f32),
                       jax.ShapeDtypeStruct((N_NODES, DIM), f32)],
        )(part[:N_NODES, :], cnt_acc, r2(conv_bias), h, wihT, whhT,
          r2(gru_bih), r2(gru_bhh), r2(ln_g), r2(ln_b))

    table = pl.pallas_call(
        _s2s_body,
        out_shape=jax.ShapeDtypeStruct((N_NODES, 3 * DIM), f32),
    )(out, batch.astype(jnp.int32).reshape(N_NODES, 1), s2s_Wih.T, s2s_Whh.T,
      r2(s2s_bih), r2(s2s_bhh))

    atom0 = target_index[0].astype(jnp.int32)
    atom1 = target_index[1].astype(jnp.int32)
    pair_idx = jnp.concatenate([atom0, atom1]).reshape(NW, 1, 2 * NPAIR // NW)
    g2 = _sc_gather(table, pair_idx, 1, 2 * NPAIR // NW, 3 * DIM)
    g2 = g2.reshape(2 * NPAIR, 3 * DIM)

    res = pl.pallas_call(
        _head_body,
        out_shape=jax.ShapeDtypeStruct((NPAIR, 1), f32),
    )(g2, target_class.astype(jnp.int32).reshape(NPAIR, 1),
      pr_W1, r2(pr_b1), r2(pr_g1), r2(pr_be1),
      pr_W2, r2(pr_b2), r2(pr_g2), r2(pr_be2), pr_W3, r2(pr_b3))
    return res.reshape(NPAIR)


# speedup vs baseline: 2.8780x; 1.0309x over previous
import functools

import jax
import jax.numpy as jnp
from jax import lax
from jax.experimental import pallas as pl
from jax.experimental.pallas import tpu as pltpu
from jax.experimental.pallas import tpu_sc as plsc

N_NODES = 10000
N_EDGES = 160000
NF = 128
EF = 16
DIM = 32
NG = 64
NPAIR = 1024
STEPS = 3
NOUT = 8
EPS = 1e-5

NW = 32
CH = 128
E_PAD = 163840
NCH_E = E_PAD // (NW * CH)
N_ACC = 10016


def _pre_body(x_ref, w1_ref, b1_ref, g1_ref, be1_ref, w2_ref, b2_ref,
              g2_ref, be2_ref, out_ref):
    x = x_ref[...]
    y = jnp.dot(x, w1_ref[...], preferred_element_type=jnp.float32) + b1_ref[...]
    mu = jnp.mean(y, axis=0, keepdims=True)
    var = jnp.mean((y - mu) ** 2, axis=0, keepdims=True)
    y = (y - mu) * (g1_ref[...] * lax.rsqrt(var + EPS)) + be1_ref[...]
    y = jnp.maximum(y, 0.0)
    z = jnp.dot(y, w2_ref[...], preferred_element_type=jnp.float32) + b2_ref[...]
    mu2 = jnp.mean(z, axis=0, keepdims=True)
    var2 = jnp.mean((z - mu2) ** 2, axis=0, keepdims=True)
    z = (z - mu2) * (g2_ref[...] * lax.rsqrt(var2 + EPS)) + be2_ref[...]
    out_ref[...] = jnp.maximum(z, 0.0)


def _enc_stats_body(ea_ref, w1_ref, b1_ref, sum_ref, sq_ref):
    @pl.when(pl.program_id(0) == 0)
    def _():
        sum_ref[...] = jnp.zeros_like(sum_ref)
        sq_ref[...] = jnp.zeros_like(sq_ref)
    y = jnp.dot(ea_ref[...], w1_ref[...], preferred_element_type=jnp.float32) + b1_ref[...]
    sum_ref[...] += jnp.sum(y, axis=0, keepdims=True)
    sq_ref[...] += jnp.sum(y * y, axis=0, keepdims=True)


def _enc_he_body(ea_ref, w1_ref, b1_ref, mu_ref, sc_ref, be_ref,
                 he_ref, hsum_ref, hh_ref):
    @pl.when(pl.program_id(0) == 0)
    def _():
        hsum_ref[...] = jnp.zeros_like(hsum_ref)
        hh_ref[...] = jnp.zeros_like(hh_ref)
    y = jnp.dot(ea_ref[...], w1_ref[...], preferred_element_type=jnp.float32) + b1_ref[...]
    he = jnp.maximum((y - mu_ref[...]) * sc_ref[...] + be_ref[...], 0.0)
    he_ref[...] = he
    hsum_ref[...] += jnp.sum(he, axis=0, keepdims=True)
    hh_ref[...] += lax.dot_general(he, he, (((0,), (0,)), ((), ())),
                                   preferred_element_type=jnp.float32,
                                   precision=lax.Precision.HIGHEST)


def _msg_body(he_ref, x0_ref, wcat_ref, amat_ref, rmat_ref, smat_ref,
              lin0_ref, lin1_ref, lin1b_ref, msg_ref):
    x0 = x0_ref[...]
    he = he_ref[...]
    u = jnp.dot(x0, wcat_ref[...], preferred_element_type=jnp.float32)
    hrep = jnp.dot(he, rmat_ref[...], preferred_element_type=jnp.float32)
    x1 = jnp.dot(hrep * u, smat_ref[...], preferred_element_type=jnp.float32)
    x1 = x1 + jnp.dot(x0, amat_ref[...], preferred_element_type=jnp.float32)
    coeff = jax.nn.sigmoid(
        jnp.dot(x0, lin0_ref[...], preferred_element_type=jnp.float32)
        + jnp.dot(x1, lin1_ref[...], preferred_element_type=jnp.float32)
        + lin1b_ref[...])
    msg_ref[...] = x1 + coeff * (x0 - x1)


def _gru_body(part_ref, cnt_ref, cb_ref, h_ref, wih_ref, whh_ref, bih_ref,
              bhh_ref, lng_ref, lnb_ref, hout_ref, out_ref):
    inv = 1.0 / jnp.maximum(cnt_ref[...][:, :1], 1.0)
    agg = part_ref[...] * inv + cb_ref[...]
    m = jnp.maximum(agg, 0.0)
    h = h_ref[...]
    gi = jnp.dot(m, wih_ref[...], preferred_element_type=jnp.float32) + bih_ref[...]
    gh = jnp.dot(h, whh_ref[...], preferred_element_type=jnp.float32) + bhh_ref[...]
    r = jax.nn.sigmoid(gi[:, :DIM] + gh[:, :DIM])
    z = jax.nn.sigmoid(gi[:, DIM:2 * DIM] + gh[:, DIM:2 * DIM])
    n = jnp.tanh(gi[:, 2 * DIM:] + r * gh[:, 2 * DIM:])
    h2 = (1.0 - z) * n + z * h
    hout_ref[...] = h2
    mu = jnp.mean(h2, axis=-1, keepdims=True)
    var = jnp.mean((h2 - mu) ** 2, axis=-1, keepdims=True)
    out_ref[...] = (h2 - mu) * lax.rsqrt(var + EPS) * lng_ref[...] + lnb_ref[...]


def _s2s_body(out_ref, batch_ref, wih_ref, whh_ref, bih_ref, bhh_ref,
              tab_ref):
    out = out_ref[...]
    b = batch_ref[...]
    giota = lax.broadcasted_iota(jnp.int32, (N_NODES, NG), 1)
    maskb = giota == b
    maskt = maskb.astype(jnp.float32)
    q = jnp.zeros((NG, 2 * DIM), jnp.float32)
    hh = jnp.zeros((NG, DIM), jnp.float32)
    cc = jnp.zeros((NG, DIM), jnp.float32)
    for _ in range(STEPS):
        g = (jnp.dot(q, wih_ref[...], preferred_element_type=jnp.float32)
             + bih_ref[...]
             + jnp.dot(hh, whh_ref[...], preferred_element_type=jnp.float32)
             + bhh_ref[...])
        i = jax.nn.sigmoid(g[:, :DIM])
        f = jax.nn.sigmoid(g[:, DIM:2 * DIM])
        gg = jnp.tanh(g[:, 2 * DIM:3 * DIM])
        o = jax.nn.sigmoid(g[:, 3 * DIM:])
        cc = f * cc + i * gg
        hh = o * jnp.tanh(cc)
        hb = jnp.dot(maskt, hh, preferred_element_type=jnp.float32)
        e = jnp.sum(out * hb, axis=-1, keepdims=True)
        em = jnp.max(jnp.where(maskb, e, -1e30), axis=0, keepdims=True)
        emb = jnp.sum(maskt * em, axis=-1, keepdims=True)
        a = jnp.exp(e - emb)
        asum = jnp.sum(maskt * a, axis=0, keepdims=True)
        asb = jnp.sum(maskt * asum, axis=-1, keepdims=True)
        an = a / (asb + 1e-16)
        r = lax.dot_general(maskt, an * out, (((0,), (0,)), ((), ())),
                            preferred_element_type=jnp.float32)
        q = jnp.concatenate([hh, r], axis=-1)
    tab_ref[:, :DIM] = out
    tab_ref[:, DIM:] = jnp.dot(maskt, q, preferred_element_type=jnp.float32)


def _head_body(g2_ref, tc_ref, w1_ref, b1_ref, g1_ref, be1_ref, w2_ref,
               b2_ref, g2w_ref, be2_ref, w3_ref, b3_ref, res_ref):
    node0 = g2_ref[:NPAIR, :DIM]
    s2s0 = g2_ref[:NPAIR, DIM:]
    node1 = g2_ref[NPAIR:, :DIM]
    feat = jnp.concatenate([node0, node1, s2s0], axis=-1)

    def ln_relu(v, g, be):
        mu = jnp.mean(v, axis=-1, keepdims=True)
        var = jnp.mean((v - mu) ** 2, axis=-1, keepdims=True)
        return jnp.maximum((v - mu) * lax.rsqrt(var + EPS) * g + be, 0.0)

    z = ln_relu(jnp.dot(feat, w1_ref[...], preferred_element_type=jnp.float32)
                + b1_ref[...], g1_ref[...], be1_ref[...])
    z = ln_relu(jnp.dot(z, w2_ref[...], preferred_element_type=jnp.float32)
                + b2_ref[...], g2w_ref[...], be2_ref[...])
    pred = jnp.dot(z, w3_ref[...], preferred_element_type=jnp.float32) + b3_ref[...]
    sel = lax.broadcasted_iota(jnp.int32, (NPAIR, NOUT), 1) == tc_ref[...]
    res_ref[...] = jnp.sum(jnp.where(sel, pred, 0.0), axis=-1, keepdims=True)


def _sc_gather(table, idxp, nch, ch, d, nround=1):
    cpr = nch // nround
    mesh = plsc.VectorSubcoreMesh(core_axis_name="c", subcore_axis_name="s")

    @functools.partial(
        pl.kernel, mesh=mesh,
        out_type=jax.ShapeDtypeStruct((NW, nch * ch, d), jnp.float32),
        compiler_params=pltpu.CompilerParams(use_tc_tiling_on_sc=False),
        scratch_types=[
            pltpu.VMEM((nch, ch), jnp.int32),
            pltpu.VMEM((cpr * ch, d), jnp.float32),
            pltpu.SemaphoreType.DMA,
        ],
    )
    def k(table_hbm, idx_hbm, out_hbm, idx_v, rows_v, sem):
        c = lax.axis_index("c")
        s = lax.axis_index("s")
        w = s * 2 + c
        pltpu.sync_copy(idx_hbm.at[w], idx_v)
        for r in range(nround):
            copies = []
            for j in range(cpr):
                copies.append(pltpu.async_copy(
                    table_hbm.at[idx_v.at[r * cpr + j]],
                    rows_v.at[pl.ds(j * ch, ch)], sem))
            for cp in copies:
                cp.wait()
            pltpu.sync_copy(rows_v,
                            out_hbm.at[w].at[pl.ds(r * cpr * ch, cpr * ch)])

    return k(table, idxp)


def _sc_scatter_add(msgp, idxp, zeros_acc, nch, ch, d, nround=1):
    cpr = nch // nround
    mesh = plsc.VectorSubcoreMesh(core_axis_name="c", subcore_axis_name="s")
    zrows = N_ACC // 16

    @functools.partial(
        pl.kernel, mesh=mesh,
        out_type=jax.ShapeDtypeStruct((N_ACC, d), jnp.float32),
        compiler_params=pltpu.CompilerParams(use_tc_tiling_on_sc=False),
        scratch_types=[
            pltpu.VMEM((2 * nch, ch), jnp.int32),
            pltpu.VMEM((2, cpr * ch, d), jnp.float32),
            pltpu.VMEM_SHARED((N_ACC, d), jnp.float32),
            pltpu.SemaphoreType.DMA,
        ],
    )
    def k(msg_hbm, idx_hbm, z_hbm, out_hbm, idx_v, msg_v, acc_sh, sem):
        c = lax.axis_index("c")
        s = lax.axis_index("s")

        @pl.when(c == 0)
        def _():
            pltpu.sync_copy(z_hbm.at[pl.ds(s * zrows, zrows)],
                            acc_sh.at[pl.ds(s * zrows, zrows)])
            plsc.subcore_barrier()
            pltpu.sync_copy(idx_hbm.at[2 * s], idx_v.at[pl.ds(0, nch)])
            pltpu.sync_copy(idx_hbm.at[2 * s + 1], idx_v.at[pl.ds(nch, nch)])
            pltpu.sync_copy(msg_hbm.at[2 * s].at[pl.ds(0, cpr * ch)],
                            msg_v.at[0])
            for rr in range(2 * nround):
                half, r = divmod(rr, nround)
                adds = []
                for j in range(cpr):
                    adds.append(pltpu.async_copy(
                        msg_v.at[rr % 2].at[pl.ds(j * ch, ch)],
                        acc_sh.at[idx_v.at[half * nch + r * cpr + j]],
                        sem, add=True))
                if rr + 1 < 2 * nround:
                    h2, r2 = divmod(rr + 1, nround)
                    pltpu.sync_copy(
                        msg_hbm.at[2 * s + h2].at[pl.ds(r2 * cpr * ch,
                                                        cpr * ch)],
                        msg_v.at[(rr + 1) % 2])
                for cp in adds:
                    cp.wait()
            plsc.subcore_barrier()
            pltpu.sync_copy(acc_sh.at[pl.ds(s * zrows, zrows)],
                            out_hbm.at[pl.ds(s * zrows, zrows)])

    return k(msgp, idxp, zeros_acc)


def _sc_count(idxp, zeros_acc, nch, ch):
    d = 16
    mesh = plsc.VectorSubcoreMesh(core_axis_name="c", subcore_axis_name="s")
    zrows = N_ACC // 16

    @functools.partial(
        pl.kernel, mesh=mesh,
        out_type=jax.ShapeDtypeStruct((N_ACC, d), jnp.float32),
        compiler_params=pltpu.CompilerParams(use_tc_tiling_on_sc=False),
        scratch_types=[
            pltpu.VMEM((2 * nch, ch), jnp.int32),
            pltpu.VMEM((ch, d), jnp.float32),
            pltpu.VMEM_SHARED((N_ACC, d), jnp.float32),
            pltpu.SemaphoreType.DMA,
        ],
    )
    def k(idx_hbm, z_hbm, out_hbm, idx_v, ones_v, acc_sh, sem):
        c = lax.axis_index("c")
        s = lax.axis_index("s")

        @pl.when(c == 0)
        def _():
            pltpu.sync_copy(z_hbm.at[pl.ds(s * zrows, zrows)],
                            acc_sh.at[pl.ds(s * zrows, zrows)])

            def fill(i, _):
                ones_v[i, :] = jnp.ones((d,), jnp.float32)
                return 0

            lax.fori_loop(0, ch, fill, 0)
            plsc.subcore_barrier()
            pltpu.sync_copy(idx_hbm.at[2 * s], idx_v.at[pl.ds(0, nch)])
            pltpu.sync_copy(idx_hbm.at[2 * s + 1], idx_v.at[pl.ds(nch, nch)])
            for j in range(2 * nch):
                pltpu.sync_copy(ones_v, acc_sh.at[idx_v.at[j]], add=True)
            plsc.subcore_barrier()
            pltpu.sync_copy(acc_sh.at[pl.ds(s * zrows, zrows)],
                            out_hbm.at[pl.ds(s * zrows, zrows)])

    return k(idxp, zeros_acc)


def kernel(x, edge_attr, edge_index, target_index, batch, target_class, pre_W1, pre_b1, pre_g1, pre_be1, pre_W2, pre_b2, pre_g2, pre_be2, enc_W1, enc_b1, enc_g1, enc_be1, enc_W2, enc_b2, enc_g2, enc_be2, lin0_W, lin1_W, lin1_b, conv_bias, gru_Wih, gru_Whh, gru_bih, gru_bhh, ln_g, ln_b, s2s_Wih, s2s_Whh, s2s_bih, s2s_bhh, pr_W1, pr_b1, pr_g1, pr_be1, pr_W2, pr_b2, pr_g2, pr_be2, pr_W3, pr_b3):
    f32 = jnp.float32
    r2 = lambda v: v.reshape(1, -1).astype(f32)
    src = edge_index[0].astype(jnp.int32)
    dst = edge_index[1].astype(jnp.int32)

    out0 = pl.pallas_call(
        _pre_body,
        out_shape=jax.ShapeDtypeStruct((N_NODES, DIM), f32),
    )(x, pre_W1, r2(pre_b1), r2(pre_g1), r2(pre_be1),
      pre_W2, r2(pre_b2), r2(pre_g2), r2(pre_be2))

    ET2 = 8000
    T2 = N_EDGES // ET2
    sum1, sq1 = pl.pallas_call(
        _enc_stats_body,
        grid=(T2,),
        in_specs=[pl.BlockSpec((ET2, EF), lambda i: (i, 0)),
                  pl.BlockSpec((EF, DIM), lambda i: (0, 0)),
                  pl.BlockSpec((1, DIM), lambda i: (0, 0))],
        out_specs=[pl.BlockSpec((1, DIM), lambda i: (0, 0)),
                   pl.BlockSpec((1, DIM), lambda i: (0, 0))],
        out_shape=[jax.ShapeDtypeStruct((1, DIM), f32),
                   jax.ShapeDtypeStruct((1, DIM), f32)],
    )(edge_attr, enc_W1, r2(enc_b1))
    mu1 = sum1 / N_EDGES
    var1 = sq1 / N_EDGES - mu1 * mu1
    sc1 = r2(enc_g1) * lax.rsqrt(var1 + EPS)

    he, hsum, hth = pl.pallas_call(
        _enc_he_body,
        grid=(T2,),
        in_specs=[pl.BlockSpec((ET2, EF), lambda i: (i, 0)),
                  pl.BlockSpec((EF, DIM), lambda i: (0, 0)),
                  pl.BlockSpec((1, DIM), lambda i: (0, 0)),
                  pl.BlockSpec((1, DIM), lambda i: (0, 0)),
                  pl.BlockSpec((1, DIM), lambda i: (0, 0)),
                  pl.BlockSpec((1, DIM), lambda i: (0, 0))],
        out_specs=[pl.BlockSpec((ET2, DIM), lambda i: (i, 0)),
                   pl.BlockSpec((1, DIM), lambda i: (0, 0)),
                   pl.BlockSpec((DIM, DIM), lambda i: (0, 0))],
        out_shape=[jax.ShapeDtypeStruct((N_EDGES, DIM), f32),
                   jax.ShapeDtypeStruct((1, DIM), f32),
                   jax.ShapeDtypeStruct((DIM, DIM), f32)],
    )(edge_attr, enc_W1, r2(enc_b1), mu1, sc1, r2(enc_be1))

    hi = lax.Precision.HIGHEST
    mh = hsum / N_EDGES
    cov = (hth / N_EDGES
           - jnp.dot(mh.T, mh, precision=hi))
    mu2 = jnp.dot(mh, enc_W2, precision=hi) + enc_b2[None, :]
    var2 = jnp.sum(enc_W2 * jnp.dot(cov, enc_W2, precision=hi),
                   axis=0, keepdims=True)
    s2 = enc_g2[None, :] * lax.rsqrt(var2 + EPS)
    a_vec = (enc_b2[None, :] - mu2) * s2 + enc_be2[None, :]
    w2s = enc_W2 * s2
    wcat = w2s.reshape(DIM, DIM, DIM).transpose(1, 0, 2).reshape(DIM, DIM * DIM)
    a_mat = a_vec.reshape(DIM, DIM)
    kk = jnp.arange(DIM * DIM, dtype=jnp.int32)
    r_mat = (jnp.arange(DIM, dtype=jnp.int32)[:, None] == kk[None, :] // DIM)
    r_mat = r_mat.astype(f32)
    s_mat = (kk[:, None] % DIM ==
             jnp.arange(DIM, dtype=jnp.int32)[None, :]).astype(f32)
    eye4 = jnp.eye(4, dtype=f32)
    wcat4 = jnp.kron(eye4, wcat)
    rmat4 = jnp.kron(eye4, r_mat)
    smat4 = jnp.kron(eye4, s_mat)
    amat4 = jnp.kron(eye4, a_mat)
    lin0_4 = jnp.kron(eye4, lin0_W)
    lin1_4 = jnp.kron(eye4, lin1_W)
    lin1b4 = jnp.tile(lin1_b, 4)[None, :]

    src_p = jnp.pad(src, (0, E_PAD - N_EDGES)).reshape(NW, NCH_E, CH)
    dst_p = jnp.pad(dst, (0, E_PAD - N_EDGES),
                    constant_values=N_NODES).reshape(NW, NCH_E, CH)
    zeros_acc = jnp.zeros((N_ACC, DIM), f32)
    zeros_acc16 = jnp.zeros((N_ACC, 16), f32)

    cnt_acc = _sc_count(dst_p, zeros_acc16, NCH_E, CH)[:N_NODES, :]

    EP4 = E_PAD // 4
    ET3 = 1024
    T3 = EP4 // ET3
    D4 = 4 * DIM
    he4 = jnp.pad(he, ((0, E_PAD - N_EDGES), (0, 0))).reshape(EP4, D4)
    h = out0
    out = out0
    wihT = gru_Wih.T
    whhT = gru_Whh.T
    for _ in range(STEPS):
        xj0 = _sc_gather(out, src_p, NCH_E, CH, DIM, nround=2).reshape(EP4, D4)
        msgp = pl.pallas_call(
            _msg_body,
            grid=(T3,),
            in_specs=[pl.BlockSpec((ET3, D4), lambda i: (i, 0)),
                      pl.BlockSpec((ET3, D4), lambda i: (i, 0)),
                      pl.BlockSpec((D4, 4 * DIM * DIM), lambda i: (0, 0)),
                      pl.BlockSpec((D4, D4), lambda i: (0, 0)),
                      pl.BlockSpec((D4, 4 * DIM * DIM), lambda i: (0, 0)),
                      pl.BlockSpec((4 * DIM * DIM, D4), lambda i: (0, 0)),
                      pl.BlockSpec((D4, D4), lambda i: (0, 0)),
                      pl.BlockSpec((D4, D4), lambda i: (0, 0)),
                      pl.BlockSpec((1, D4), lambda i: (0, 0))],
            out_specs=pl.BlockSpec((ET3, D4), lambda i: (i, 0)),
            out_shape=jax.ShapeDtypeStruct((EP4, D4), f32),
        )(he4, xj0, wcat4, amat4, rmat4, smat4, lin0_4, lin1_4, lin1b4)
        part = _sc_scatter_add(msgp.reshape(NW, NCH_E * CH, DIM), dst_p,
                               zeros_acc, NCH_E, CH, DIM, nround=4)
        h, out = pl.pallas_call(
            _gru_body,
            out_shape=[jax.ShapeDtypeStruct((N_NODES, DIM), f32),
                       jax.ShapeDtypeStruct((N_NODES, DIM), f32)],
        )(part[:N_NODES, :], cnt_acc, r2(conv_bias), h, wihT, whhT,
          r2(gru_bih), r2(gru_bhh), r2(ln_g), r2(ln_b))

    table = pl.pallas_call(
        _s2s_body,
        out_shape=jax.ShapeDtypeStruct((N_NODES, 3 * DIM), f32),
    )(out, batch.astype(jnp.int32).reshape(N_NODES, 1), s2s_Wih.T, s2s_Whh.T,
      r2(s2s_bih), r2(s2s_bhh))

    atom0 = target_index[0].astype(jnp.int32)
    atom1 = target_index[1].astype(jnp.int32)
    pair_idx = jnp.concatenate([atom0, atom1]).reshape(NW, 1, 2 * NPAIR // NW)
    g2 = _sc_gather(table, pair_idx, 1, 2 * NPAIR // NW, 3 * DIM)
    g2 = g2.reshape(2 * NPAIR, 3 * DIM)

    res = pl.pallas_call(
        _head_body,
        out_shape=jax.ShapeDtypeStruct((NPAIR, 1), f32),
    )(g2, target_class.astype(jnp.int32).reshape(NPAIR, 1),
      pr_W1, r2(pr_b1), r2(pr_g1), r2(pr_be1),
      pr_W2, r2(pr_b2), r2(pr_g2), r2(pr_be2), pr_W3, r2(pr_b3))
    return res.reshape(NPAIR)
